# baseline (speedup 1.0000x reference)
import functools

import jax
import jax.numpy as jnp
from jax import lax
from jax.experimental import pallas as pl
from jax.experimental.pallas import tpu as pltpu
from jax.experimental.pallas import tpu_sc as plsc

F32 = jnp.float32
H = 32
G = 32
KS = 15
S0 = 2.0


def _gather_rows(table, idx):
    V, D = table.shape
    B = idx.shape[0]
    info = plsc.get_sparse_core_info()
    nw = info.num_cores * info.num_subcores
    ch_max = min(128, (65536 // D) // 8 * 8)
    r = -(-B // nw)
    if r <= ch_max:
        ch = max(8, -(-r // 8) * 8)
        r = ch
    else:
        ch = ch_max
        r = -(-r // ch) * ch
    bp = r * nw
    nch = r // ch
    idx = jnp.pad(idx.astype(jnp.int32).reshape(-1), (0, bp - B))

    mesh = plsc.VectorSubcoreMesh(core_axis_name="c", subcore_axis_name="s")

    @functools.partial(
        pl.kernel, mesh=mesh,
        compiler_params=pltpu.CompilerParams(use_tc_tiling_on_sc=False),
        out_type=jax.ShapeDtypeStruct((bp, D), F32),
        scratch_types=[
            pltpu.VMEM((ch,), jnp.int32),
            pltpu.VMEM((ch, D), F32),
            pltpu.SemaphoreType.DMA,
        ],
    )
    def gk(table_hbm, idx_hbm, out_hbm, idx_v, rows_v, sem):
        wid = lax.axis_index("s") * info.num_cores + lax.axis_index("c")
        base = wid * r

        def one(start):
            pltpu.sync_copy(idx_hbm.at[pl.ds(start, ch)], idx_v)
            pltpu.async_copy(table_hbm.at[idx_v], rows_v, sem).wait()
            pltpu.sync_copy(rows_v, out_hbm.at[pl.ds(start, ch)])

        if nch <= 8:
            for j in range(nch):
                one(pl.multiple_of(base + j * ch, 8))
        else:
            def body(j, carry):
                one(pl.multiple_of(base + j * ch, 8))
                return carry
            lax.fori_loop(0, nch, body, 0)

    return gk(table, idx)[:B]


def _leaky(x):
    return jnp.maximum(x, 0.1 * x)


def _gn_vals(t, g, be, ig, igt, n):
    c = t.shape[1]
    cnt = n * (c // G)
    m = (jnp.sum(t, axis=0, keepdims=True) @ ig) * (1.0 / cnt)
    mc = m @ igt
    tc = t - mc
    v = (jnp.sum(tc * tc, axis=0, keepdims=True) @ ig) * (1.0 / cnt)
    inv = lax.rsqrt(v + 1e-5) @ igt
    return tc * inv * g + be


def _ig_mats(c):
    a = (jnp.arange(c)[:, None] // (c // G) == jnp.arange(G)[None, :])
    ig = a.astype(F32)
    return ig, ig.T


def _row2(x):
    return x.reshape(1, -1)


def _pad_rows(x, n_pad):
    n = x.shape[0]
    if n_pad == n:
        return x
    return jnp.pad(x, ((0, n_pad - n), (0, 0)))


def _pad3(x, n_pad):
    n = x.shape[0]
    if n_pad == n:
        return x
    return jnp.pad(x, ((0, n_pad - n), (0, 0), (0, 0)))


def _pad16(pts):
    n = pts.shape[0]
    return jnp.concatenate([pts, jnp.zeros((n, 13), F32)], axis=1)


def _unary2(x1, x2, W1, W2, b, g=None, be=None, relu=True):
    n = x1.shape[0]
    cout = W1.shape[1]
    gn = g is not None
    if gn:
        ig, igt = _ig_mats(cout)

    def body(*refs):
        if gn:
            x1_r, x2_r, w1_r, w2_r, b_r, g_r, be_r, ig_r, igt_r, o_r = refs
        else:
            x1_r, x2_r, w1_r, w2_r, b_r, o_r = refs
        t = (jnp.dot(x1_r[...], w1_r[...], preferred_element_type=F32)
             + jnp.dot(x2_r[...], w2_r[...], preferred_element_type=F32)
             + b_r[...])
        if gn:
            t = _gn_vals(t, g_r[...], be_r[...], ig_r[...], igt_r[...], n)
        if relu:
            t = _leaky(t)
        o_r[...] = t

    args = [x1, x2, W1, W2, _row2(b)]
    if gn:
        args += [_row2(g), _row2(be), ig, igt]
    return pl.pallas_call(
        body, out_shape=jax.ShapeDtypeStruct((n, cout), F32))(*args)


def _block(npf, nff, qp16, kp, w, sigma, tailp, scf=None, mxg=None,
           u1p=None, scp=None):
    n = qp16.shape[0]
    c = nff.shape[2]
    mid = w.shape[2]
    gn_g, gn_be, Wu2, bu2, gu2, beu2 = tailp
    cout = Wu2.shape[1]
    cp = max(c, 128) + 128 + (max(cout, 128) if mxg is not None else 0)
    q = max(8, min(512, (4_000_000 // (H * cp * 4)) // 8 * 8))
    if c <= 64 and n > 2500:
        q = min(q, 80)
    if n <= q:
        q = -(-n // 8) * 8
    n_pad = -(-n // q) * q
    grid = n_pad // q
    npf = _pad3(npf, n_pad)
    nff = _pad3(nff, n_pad)
    qp16 = _pad_rows(qp16, n_pad)
    if mxg is not None:
        mxg = _pad3(mxg, n_pad)
    kpx = jnp.full((3, 16), 1e6, F32).at[:, :KS].set(kp.T)
    inv_sig = 1.0 / sigma
    small = c <= 64

    igm = _ig_mats(mid)
    igc = _ig_mats(cout)
    args = [npf, nff, qp16, kpx]
    specs = [
        pl.BlockSpec((q, H, 16), lambda i: (i, 0, 0)),
        pl.BlockSpec((q, H, c), lambda i: (i, 0, 0)),
        pl.BlockSpec((q, 16), lambda i: (i, 0)),
        pl.BlockSpec((3, 16), lambda i: (0, 0)),
    ]
    if small:
        kc = jnp.arange(KS * c)
        e1 = (kc[None, :] // c == jnp.arange(16)[:, None]).astype(F32)
        e2 = (kc[None, :] % c == jnp.arange(c)[:, None]).astype(F32)
        args += [w.reshape(KS * c, mid), e1, e2]
        specs += [pl.BlockSpec((KS * c, mid), lambda i: (0, 0)),
                  pl.BlockSpec((16, KS * c), lambda i: (0, 0)),
                  pl.BlockSpec((c, KS * c), lambda i: (0, 0))]
    else:
        args += [w]
        specs += [pl.BlockSpec((KS, c, mid), lambda i: (0, 0, 0))]

    def full2(a):
        a = jnp.asarray(a)
        specs.append(pl.BlockSpec(a.shape, lambda i: (0,) * a.ndim))
        args.append(a)

    if mxg is not None:
        specs.append(pl.BlockSpec((q, H, cout), lambda i: (i, 0, 0)))
        args.append(mxg)
    else:
        full2(scf)
    for a in (_row2(gn_g), _row2(gn_be), Wu2, _row2(bu2), _row2(gu2),
              _row2(beu2), igm[0], igm[1], igc[0], igc[1]):
        full2(a)
    outs = [jax.ShapeDtypeStruct((n, cout), F32)]
    out_specs = [pl.BlockSpec((n, cout), lambda i: (0, 0))]
    if u1p is not None:
        w1, b1, g1, be1 = u1p
        ig1 = _ig_mats(w1.shape[1])
        for a in (w1, _row2(b1), _row2(g1), _row2(be1), ig1[0], ig1[1]):
            full2(a)
        outs.append(jax.ShapeDtypeStruct((n, w1.shape[1]), F32))
        out_specs.append(pl.BlockSpec((n, w1.shape[1]), lambda i: (0, 0)))
    if scp is not None:
        w2, b2, g2, be2 = scp
        ig2 = _ig_mats(w2.shape[1])
        for a in (w2, _row2(b2), _row2(g2), _row2(be2), ig2[0], ig2[1]):
            full2(a)
        outs.append(jax.ShapeDtypeStruct((n, w2.shape[1]), F32))
        out_specs.append(pl.BlockSpec((n, w2.shape[1]), lambda i: (0, 0)))

    u1_ws = u1p[0].shape[1] if u1p is not None else None
    sc_ws = scp[0].shape[1] if scp is not None else None
    scratch = [pltpu.VMEM((n_pad, mid), F32),
               pltpu.VMEM((n_pad, cout), F32)]
    if mxg is not None:
        scratch.append(pltpu.VMEM((n_pad, cout), F32))
    if u1p is not None:
        scratch.append(pltpu.VMEM((n_pad, u1_ws), F32))
    if scp is not None:
        scratch.append(pltpu.VMEM((n_pad, sc_ws), F32))

    n_in = len(args)
    n_out = len(outs)

    def body(*refs):
        i = pl.program_id(0)
        pos = 4
        np_r, nf_r, qp_r, kpx_r = refs[:4]
        if small:
            wf_r, e1_r, e2_r = refs[pos:pos + 3]
            pos += 3
        else:
            wf_r = refs[pos]
            pos += 1
        if mxg is not None:
            mx_r = refs[pos]
        else:
            scf_r = refs[pos]
        pos += 1
        (gn_g_r, gn_be_r, wu2_r, bu2_r, gu2_r, beu2_r,
         igm0_r, igm1_r, igc0_r, igc1_r) = refs[pos:pos + 10]
        pos += 10
        if u1p is not None:
            u1_rs = refs[pos:pos + 6]
            pos += 6
        if scp is not None:
            sc_rs = refs[pos:pos + 6]
            pos += 6
        orefs = refs[n_in:n_in + n_out]
        spos = n_in + n_out
        acc_r = refs[spos]
        u2s_r = refs[spos + 1]
        spos += 2
        if mxg is not None:
            mxs_r = refs[spos]
            spos += 1
        if u1p is not None:
            t1s_r = refs[spos]
            spos += 1
        if scp is not None:
            t2s_r = refs[spos]

        qpb = qp_r[...]
        kx = kpx_r[0:1, :]
        ky = kpx_r[1:2, :]
        kz = kpx_r[2:3, :]

        def weights(h):
            rel = np_r[:, h, :] - qpb
            dx = rel[:, 0:1] - kx
            dy = rel[:, 1:2] - ky
            dz = rel[:, 2:3] - kz
            d2 = dx * dx + dy * dy + dz * dz
            return jnp.maximum(1.0 - jnp.sqrt(d2) * inv_sig, 0.0)

        cnt = None
        for h in range(H):
            ns = (jnp.sum(nf_r[:, h, :], axis=1, keepdims=True) > 0.0
                  ).astype(F32)
            cnt = ns if cnt is None else cnt + ns
        if small:
            A = None
            for h in range(H):
                ww = jnp.dot(weights(h), e1_r[...],
                             preferred_element_type=F32)
                nw = jnp.dot(nf_r[:, h, :], e2_r[...],
                             preferred_element_type=F32)
                u = ww * nw
                A = u if A is None else A + u
            acc = jnp.dot(A, wf_r[...], preferred_element_type=F32)
        else:
            wlist = [weights(h) for h in range(H)]
            acc = None
            for k in range(KS):
                ak = None
                for h in range(H):
                    u = wlist[h][:, k:k + 1] * nf_r[:, h, :]
                    ak = u if ak is None else ak + u
                pk = jnp.dot(ak, wf_r[k], preferred_element_type=F32)
                acc = pk if acc is None else acc + pk
        acc_r[pl.ds(i * q, q), :] = acc / jnp.maximum(cnt, 1.0)
        if mxg is not None:
            m = mx_r[:, 0, :]
            for h in range(1, H):
                m = jnp.maximum(m, mx_r[:, h, :])
            mxs_r[pl.ds(i * q, q), :] = m

        @pl.when(i == grid - 1)
        def _tail():
            full = n // q
            rem = n - full * q

            def gn_coefs(s1, s2, ig0_v, ig1_v, cnt_, g_v, be_v):
                m_ = jnp.dot(s1, ig0_v, preferred_element_type=F32) / cnt_
                v = (jnp.dot(s2, ig0_v, preferred_element_type=F32) / cnt_
                     - m_ * m_)
                inv = lax.rsqrt(v + 1e-5)
                mc = jnp.dot(m_, ig1_v, preferred_element_type=F32)
                invc = jnp.dot(inv, ig1_v, preferred_element_type=F32)
                mult = invc * g_v
                return mult, be_v - mc * mult

            def p1(j, cy):
                t = acc_r[pl.ds(j * q, q), :]
                return (cy[0] + jnp.sum(t, 0, keepdims=True),
                        cy[1] + jnp.sum(t * t, 0, keepdims=True))
            s1, s2 = lax.fori_loop(0, full, p1,
                                   (jnp.zeros((1, mid), F32),
                                    jnp.zeros((1, mid), F32)))
            if rem:
                t = acc_r[pl.ds(full * q, rem), :]
                s1 += jnp.sum(t, 0, keepdims=True)
                s2 += jnp.sum(t * t, 0, keepdims=True)
            mult1, add1 = gn_coefs(s1, s2, igm0_r[...], igm1_r[...],
                                   n * (mid // G), gn_g_r[...], gn_be_r[...])

            def p2c(j, sz):
                t = _leaky(acc_r[pl.ds(j * q, sz), :] * mult1 + add1)
                return jnp.dot(t, wu2_r[...],
                               preferred_element_type=F32) + bu2_r[...]

            def p2(j, cy):
                u = p2c(j, q)
                u2s_r[pl.ds(j * q, q), :] = u
                return (cy[0] + jnp.sum(u, 0, keepdims=True),
                        cy[1] + jnp.sum(u * u, 0, keepdims=True))
            s1, s2 = lax.fori_loop(0, full, p2,
                                   (jnp.zeros((1, cout), F32),
                                    jnp.zeros((1, cout), F32)))
            if rem:
                u = p2c(full, rem)
                u2s_r[pl.ds(full * q, rem), :] = u
                s1 += jnp.sum(u, 0, keepdims=True)
                s2 += jnp.sum(u * u, 0, keepdims=True)
            mult2, add2 = gn_coefs(s1, s2, igc0_r[...], igc1_r[...],
                                   n * (cout // G), gu2_r[...], beu2_r[...])

            nu1 = u1p is not None
            nsc = scp is not None
            c1 = u1_ws if nu1 else 8
            c2 = sc_ws if nsc else 8

            def p3body(j, sz, cy):
                u = u2s_r[pl.ds(j * q, sz), :]
                if mxg is not None:
                    sc_c = mxs_r[pl.ds(j * q, sz), :]
                else:
                    sc_c = scf_r[pl.ds(j * q, sz), :]
                f = _leaky(u * mult2 + add2 + sc_c)
                orefs[0][pl.ds(j * q, sz), :] = f
                a1, b1_, a2, b2_ = cy
                if nu1:
                    t1 = (jnp.dot(f, u1_rs[0][...],
                                  preferred_element_type=F32) + u1_rs[1][...])
                    t1s_r[pl.ds(j * q, sz), :] = t1
                    a1 = a1 + jnp.sum(t1, 0, keepdims=True)
                    b1_ = b1_ + jnp.sum(t1 * t1, 0, keepdims=True)
                if nsc:
                    t2 = (jnp.dot(f, sc_rs[0][...],
                                  preferred_element_type=F32) + sc_rs[1][...])
                    t2s_r[pl.ds(j * q, sz), :] = t2
                    a2 = a2 + jnp.sum(t2, 0, keepdims=True)
                    b2_ = b2_ + jnp.sum(t2 * t2, 0, keepdims=True)
                return (a1, b1_, a2, b2_)

            cy0 = (jnp.zeros((1, c1), F32), jnp.zeros((1, c1), F32),
                   jnp.zeros((1, c2), F32), jnp.zeros((1, c2), F32))
            cy = lax.fori_loop(0, full, lambda j, cy: p3body(j, q, cy), cy0)
            if rem:
                cy = p3body(full, rem, cy)
            a1, b1_, a2, b2_ = cy

            oi = 1
            if nu1:
                mu, au = gn_coefs(a1, b1_, u1_rs[4][...], u1_rs[5][...],
                                  n * (c1 // G), u1_rs[2][...], u1_rs[3][...])
                o1 = orefs[1]

                def p4(j, z):
                    o1[pl.ds(j * q, q), :] = _leaky(
                        t1s_r[pl.ds(j * q, q), :] * mu + au)
                    return z
                lax.fori_loop(0, full, p4, 0)
                if rem:
                    o1[pl.ds(full * q, rem), :] = _leaky(
                        t1s_r[pl.ds(full * q, rem), :] * mu + au)
                oi = 2
            if nsc:
                ms, as_ = gn_coefs(a2, b2_, sc_rs[4][...], sc_rs[5][...],
                                   n * (c2 // G), sc_rs[2][...], sc_rs[3][...])
                o2 = orefs[oi]

                def p5(j, z):
                    o2[pl.ds(j * q, q), :] = (
                        t2s_r[pl.ds(j * q, q), :] * ms + as_)
                    return z
                lax.fori_loop(0, full, p5, 0)
                if rem:
                    o2[pl.ds(full * q, rem), :] = (
                        t2s_r[pl.ds(full * q, rem), :] * ms + as_)

    res = pl.pallas_call(
        body,
        grid=(grid,),
        in_specs=specs,
        out_specs=out_specs,
        out_shape=outs,
        scratch_shapes=scratch)(*args)
    return res


def _first_block(npf, qp16, kp, w, sigma, g0, be0, u1p, scp):
    n = qp16.shape[0]
    d = w.shape[2]
    q = max(8, min(512, (4_000_000 // (H * 256 * 4)) // 8 * 8))
    if n <= q:
        q = -(-n // 8) * 8
    n_pad = -(-n // q) * q
    grid = n_pad // q
    npf = _pad3(npf, n_pad)
    qp16 = _pad_rows(qp16, n_pad)
    kpx = jnp.full((3, 16), 1e6, F32).at[:, :KS].set(kp.T)
    w0 = jnp.zeros((16, d), F32).at[:KS, :].set(w[:, 0, :])
    inv_sig = 1.0 / sigma
    ig0 = _ig_mats(d)
    w1, b1, g1, be1 = u1p
    ig1 = _ig_mats(w1.shape[1])
    w2, b2, g2, be2 = scp
    ig2 = _ig_mats(w2.shape[1])
    c1 = w1.shape[1]
    c2 = w2.shape[1]

    def body(np_r, qp_r, kpx_r, w0_r, g0_r, be0_r, ig00_r, ig01_r,
             w1_r, b1_r, g1_r, be1_r, ig10_r, ig11_r,
             w2_r, b2_r, g2_r, be2_r, ig20_r, ig21_r,
             o1_r, o2_r, acc_r, t1s_r, t2s_r):
        i = pl.program_id(0)
        qpb = qp_r[...]
        kx = kpx_r[0:1, :]
        ky = kpx_r[1:2, :]
        kz = kpx_r[2:3, :]
        S = None
        for h in range(H):
            rel = np_r[:, h, :] - qpb
            dx = rel[:, 0:1] - kx
            dy = rel[:, 1:2] - ky
            dz = rel[:, 2:3] - kz
            d2 = dx * dx + dy * dy + dz * dz
            wts = jnp.maximum(1.0 - jnp.sqrt(d2) * inv_sig, 0.0)
            S = wts if S is None else S + wts
        acc_r[pl.ds(i * q, q), :] = (
            jnp.dot(S, w0_r[...], preferred_element_type=F32) * (1.0 / H))

        @pl.when(i == grid - 1)
        def _tail():
            full = n // q
            rem = n - full * q

            def gn_coefs(s1, s2, ig0_v, ig1_v, cnt_, g_v, be_v):
                m_ = jnp.dot(s1, ig0_v, preferred_element_type=F32) / cnt_
                v = (jnp.dot(s2, ig0_v, preferred_element_type=F32) / cnt_
                     - m_ * m_)
                inv = lax.rsqrt(v + 1e-5)
                mc = jnp.dot(m_, ig1_v, preferred_element_type=F32)
                invc = jnp.dot(inv, ig1_v, preferred_element_type=F32)
                mult = invc * g_v
                return mult, be_v - mc * mult

            def p1(j, cy):
                t = acc_r[pl.ds(j * q, q), :]
                return (cy[0] + jnp.sum(t, 0, keepdims=True),
                        cy[1] + jnp.sum(t * t, 0, keepdims=True))
            s1, s2 = lax.fori_loop(0, full, p1,
                                   (jnp.zeros((1, d), F32),
                                    jnp.zeros((1, d), F32)))
            if rem:
                t = acc_r[pl.ds(full * q, rem), :]
                s1 += jnp.sum(t, 0, keepdims=True)
                s2 += jnp.sum(t * t, 0, keepdims=True)
            mult0, add0 = gn_coefs(s1, s2, ig00_r[...], ig01_r[...],
                                   n * (d // G), g0_r[...], be0_r[...])

            def p2body(j, sz, cy):
                f = _leaky(acc_r[pl.ds(j * q, sz), :] * mult0 + add0)
                t1 = jnp.dot(f, w1_r[...],
                             preferred_element_type=F32) + b1_r[...]
                t2 = jnp.dot(f, w2_r[...],
                             preferred_element_type=F32) + b2_r[...]
                t1s_r[pl.ds(j * q, sz), :] = t1
                t2s_r[pl.ds(j * q, sz), :] = t2
                a1, b1_, a2, b2_ = cy
                return (a1 + jnp.sum(t1, 0, keepdims=True),
                        b1_ + jnp.sum(t1 * t1, 0, keepdims=True),
                        a2 + jnp.sum(t2, 0, keepdims=True),
                        b2_ + jnp.sum(t2 * t2, 0, keepdims=True))
            cy0 = (jnp.zeros((1, c1), F32), jnp.zeros((1, c1), F32),
                   jnp.zeros((1, c2), F32), jnp.zeros((1, c2), F32))
            cy = lax.fori_loop(0, full, lambda j, cy: p2body(j, q, cy), cy0)
            if rem:
                cy = p2body(full, rem, cy)
            a1, b1_, a2, b2_ = cy

            mu, au = gn_coefs(a1, b1_, ig10_r[...], ig11_r[...],
                              n * (c1 // G), g1_r[...], be1_r[...])

            def p3(j, z):
                o1_r[pl.ds(j * q, q), :] = _leaky(
                    t1s_r[pl.ds(j * q, q), :] * mu + au)
                return z
            lax.fori_loop(0, full, p3, 0)
            if rem:
                o1_r[pl.ds(full * q, rem), :] = _leaky(
                    t1s_r[pl.ds(full * q, rem), :] * mu + au)

            ms, as_ = gn_coefs(a2, b2_, ig20_r[...], ig21_r[...],
                               n * (c2 // G), g2_r[...], be2_r[...])

            def p4(j, z):
                o2_r[pl.ds(j * q, q), :] = (
                    t2s_r[pl.ds(j * q, q), :] * ms + as_)
                return z
            lax.fori_loop(0, full, p4, 0)
            if rem:
                o2_r[pl.ds(full * q, rem), :] = (
                    t2s_r[pl.ds(full * q, rem), :] * ms + as_)

    full_spec = lambda a: pl.BlockSpec(a.shape, lambda i: (0,) * a.ndim)
    args = [npf, qp16, kpx, w0, _row2(g0), _row2(be0), ig0[0], ig0[1],
            w1, _row2(b1), _row2(g1), _row2(be1), ig1[0], ig1[1],
            w2, _row2(b2), _row2(g2), _row2(be2), ig2[0], ig2[1]]
    specs = ([pl.BlockSpec((q, H, 16), lambda i: (i, 0, 0)),
              pl.BlockSpec((q, 16), lambda i: (i, 0))]
             + [full_spec(a) for a in args[2:]])
    return pl.pallas_call(
        body,
        grid=(grid,),
        in_specs=specs,
        out_specs=[pl.BlockSpec((n, c1), lambda i: (0, 0)),
                   pl.BlockSpec((n, c2), lambda i: (0, 0))],
        out_shape=[jax.ShapeDtypeStruct((n, c1), F32),
                   jax.ShapeDtypeStruct((n, c2), F32)],
        scratch_shapes=[pltpu.VMEM((n_pad, d), F32),
                        pltpu.VMEM((n_pad, c1), F32),
                        pltpu.VMEM((n_pad, c2), F32)])(*args)


def kernel(feats, points_0, points_1, points_2, points_3, neighbors_0,
           neighbors_1, neighbors_2, neighbors_3, subsampling_0,
           subsampling_1, subsampling_2, upsampling_0, upsampling_1,
           upsampling_2, params):
    p = params

    def up(name):
        return (p[name + '_W'], p[name + '_b'], p[name + '_g'],
                p[name + '_be'])

    def tp(name):
        return (p[name + '_n_g'], p[name + '_n_be'], p[name + '_u2_W'],
                p[name + '_u2_b'], p[name + '_u2_g'], p[name + '_u2_be'])

    n1, n2, n3, n4 = (points_0.shape[0], points_1.shape[0],
                      points_2.shape[0], points_3.shape[0])
    pt0, pt1, pt2, pt3 = (_pad16(points_0), _pad16(points_1),
                          _pad16(points_2), _pad16(points_3))

    nb0 = neighbors_0.reshape(-1)
    nb1 = neighbors_1.reshape(-1)
    nb2 = neighbors_2.reshape(-1)
    nb3 = neighbors_3.reshape(-1)
    sb0 = subsampling_0.reshape(-1)
    sb1 = subsampling_1.reshape(-1)
    sb2 = subsampling_2.reshape(-1)

    pts_all = jnp.concatenate([pt0, pt1, pt2, pt3], axis=0)
    o1, o2, o3 = n1, n1 + n2, n1 + n2 + n3
    idx_all = jnp.concatenate([
        nb0, sb0, nb1 + o1, sb1 + o1, nb2 + o2, sb2 + o2, nb3 + o3])
    gp = _gather_rows(pts_all, idx_all)
    s = 0
    segs = []
    for cnt, nn in ((n1 * H, n1), (n2 * H, n2), (n2 * H, n2), (n3 * H, n3),
                    (n3 * H, n3), (n4 * H, n4), (n4 * H, n4)):
        segs.append(gp[s:s + cnt].reshape(nn, H, 16))
        s += cnt
    np_n0, np_s0, np_n1, np_s1, np_n2, np_s2, np_n3 = segs

    h12, sc12 = _first_block(np_n0, pt0, p['e11_kp'], p['e11_w'], S0,
                             p['e11_g'], p['e11_be'],
                             up('e12_u1'), up('e12_sc'))
    f1, h21 = _block(np_n0, _gather_rows(h12, nb0).reshape(n1, H, 32),
                     pt0, p['e12_kp'], p['e12_w'], S0, tp('e12'),
                     scf=sc12, u1p=up('e21_u1'))

    mxg = _gather_rows(f1, sb0).reshape(n2, H, 128)
    f2a, h22, sc22 = _block(np_s0, _gather_rows(h21, sb0).reshape(n2, H, 32),
                            pt1, p['e21_kp'], p['e21_w'], S0, tp('e21'),
                            mxg=mxg, u1p=up('e22_u1'), scp=up('e22_sc'))
    f2b, h23 = _block(np_n1, _gather_rows(h22, nb1).reshape(n2, H, 64),
                      pt1, p['e22_kp'], p['e22_w'], 2 * S0, tp('e22'),
                      scf=sc22, u1p=up('e23_u1'))
    f2, h31 = _block(np_n1, _gather_rows(h23, nb1).reshape(n2, H, 64),
                     pt1, p['e23_kp'], p['e23_w'], 2 * S0, tp('e23'),
                     scf=f2b, u1p=up('e31_u1'))

    mxg = _gather_rows(f2, sb1).reshape(n3, H, 256)
    f3a, h32, sc32 = _block(np_s1, _gather_rows(h31, sb1).reshape(n3, H, 64),
                            pt2, p['e31_kp'], p['e31_w'], 2 * S0, tp('e31'),
                            mxg=mxg, u1p=up('e32_u1'), scp=up('e32_sc'))
    f3b, h33 = _block(np_n2, _gather_rows(h32, nb2).reshape(n3, H, 128),
                      pt2, p['e32_kp'], p['e32_w'], 4 * S0, tp('e32'),
                      scf=sc32, u1p=up('e33_u1'))
    f3, h41 = _block(np_n2, _gather_rows(h33, nb2).reshape(n3, H, 128),
                     pt2, p['e33_kp'], p['e33_w'], 4 * S0, tp('e33'),
                     scf=f3b, u1p=up('e41_u1'))

    mxg = _gather_rows(f3, sb2).reshape(n4, H, 512)
    f4a, h42, sc42 = _block(np_s2, _gather_rows(h41, sb2).reshape(n4, H, 128),
                            pt3, p['e41_kp'], p['e41_w'], 4 * S0, tp('e41'),
                            mxg=mxg, u1p=up('e42_u1'), scp=up('e42_sc'))
    f4b, h43 = _block(np_n3, _gather_rows(h42, nb3).reshape(n4, H, 256),
                      pt3, p['e42_kp'], p['e42_w'], 8 * S0, tp('e42'),
                      scf=sc42, u1p=up('e43_u1'))
    (f4,) = _block(np_n3, _gather_rows(h43, nb3).reshape(n4, H, 256),
                   pt3, p['e43_kp'], p['e43_w'], 8 * S0, tp('e43'),
                   scf=f4b)

    up3 = _gather_rows(f4, upsampling_2[:, 0])
    l3 = _unary2(up3, f3, p['d3_W'][:1024], p['d3_W'][1024:], p['d3_b'],
                 p['d3_g'], p['d3_be'])
    up2 = _gather_rows(l3, upsampling_1[:, 0])
    l2 = _unary2(up2, f2, p['d2_W'][:512], p['d2_W'][512:], p['d2_b'],
                 relu=False)

    return ([l2, l3, f4], [f1, f2, f3])


# speedup vs baseline: 1.0083x; 1.0083x over previous
import functools

import jax
import jax.numpy as jnp
from jax import lax
from jax.experimental import pallas as pl
from jax.experimental.pallas import tpu as pltpu
from jax.experimental.pallas import tpu_sc as plsc

F32 = jnp.float32
H = 32
G = 32
KS = 15
S0 = 2.0


def _gather_rows(table, idx):
    V, D = table.shape
    B = idx.shape[0]
    info = plsc.get_sparse_core_info()
    nw = info.num_cores * info.num_subcores
    ch_max = min(128, (65536 // D) // 8 * 8)
    r = -(-B // nw)
    if r <= ch_max:
        ch = max(8, -(-r // 8) * 8)
        r = ch
    else:
        ch = ch_max
        r = -(-r // ch) * ch
    bp = r * nw
    nch = r // ch
    idx = jnp.pad(idx.astype(jnp.int32).reshape(-1), (0, bp - B))

    mesh = plsc.VectorSubcoreMesh(core_axis_name="c", subcore_axis_name="s")

    @functools.partial(
        pl.kernel, mesh=mesh,
        compiler_params=pltpu.CompilerParams(use_tc_tiling_on_sc=False),
        out_type=jax.ShapeDtypeStruct((bp, D), F32),
        scratch_types=[
            pltpu.VMEM((ch,), jnp.int32),
            pltpu.VMEM((ch, D), F32),
            pltpu.SemaphoreType.DMA,
        ],
    )
    def gk(table_hbm, idx_hbm, out_hbm, idx_v, rows_v, sem):
        wid = lax.axis_index("s") * info.num_cores + lax.axis_index("c")
        base = wid * r

        def one(start):
            pltpu.sync_copy(idx_hbm.at[pl.ds(start, ch)], idx_v)
            pltpu.async_copy(table_hbm.at[idx_v], rows_v, sem).wait()
            pltpu.sync_copy(rows_v, out_hbm.at[pl.ds(start, ch)])

        if nch <= 8:
            for j in range(nch):
                one(pl.multiple_of(base + j * ch, 8))
        else:
            def body(j, carry):
                one(pl.multiple_of(base + j * ch, 8))
                return carry
            lax.fori_loop(0, nch, body, 0)

    return gk(table, idx)[:B]


def _leaky(x):
    return jnp.maximum(x, 0.1 * x)


def _gn_vals(t, g, be, ig, igt, n):
    c = t.shape[1]
    cnt = n * (c // G)
    m = (jnp.sum(t, axis=0, keepdims=True) @ ig) * (1.0 / cnt)
    mc = m @ igt
    tc = t - mc
    v = (jnp.sum(tc * tc, axis=0, keepdims=True) @ ig) * (1.0 / cnt)
    inv = lax.rsqrt(v + 1e-5) @ igt
    return tc * inv * g + be


def _ig_mats(c):
    a = (jnp.arange(c)[:, None] // (c // G) == jnp.arange(G)[None, :])
    ig = a.astype(F32)
    return ig, ig.T


def _row2(x):
    return x.reshape(1, -1)


def _pad_rows(x, n_pad):
    n = x.shape[0]
    if n_pad == n:
        return x
    return jnp.pad(x, ((0, n_pad - n), (0, 0)))


def _pad3(x, n_pad):
    n = x.shape[0]
    if n_pad == n:
        return x
    return jnp.pad(x, ((0, n_pad - n), (0, 0), (0, 0)))


def _pad16(pts):
    n = pts.shape[0]
    return jnp.concatenate([pts, jnp.zeros((n, 13), F32)], axis=1)


def _unary2(x1, x2, W1, W2, b, g=None, be=None, relu=True):
    n = x1.shape[0]
    cout = W1.shape[1]
    gn = g is not None
    if gn:
        ig, igt = _ig_mats(cout)

    def body(*refs):
        if gn:
            x1_r, x2_r, w1_r, w2_r, b_r, g_r, be_r, ig_r, igt_r, o_r = refs
        else:
            x1_r, x2_r, w1_r, w2_r, b_r, o_r = refs
        t = (jnp.dot(x1_r[...], w1_r[...], preferred_element_type=F32)
             + jnp.dot(x2_r[...], w2_r[...], preferred_element_type=F32)
             + b_r[...])
        if gn:
            t = _gn_vals(t, g_r[...], be_r[...], ig_r[...], igt_r[...], n)
        if relu:
            t = _leaky(t)
        o_r[...] = t

    args = [x1, x2, W1, W2, _row2(b)]
    if gn:
        args += [_row2(g), _row2(be), ig, igt]
    return pl.pallas_call(
        body, out_shape=jax.ShapeDtypeStruct((n, cout), F32))(*args)


def _block(npf, nff, qp16, kp, w, sigma, tailp, scf=None, mxg=None,
           u1p=None, scp=None):
    n = qp16.shape[0]
    c = nff.shape[2]
    mid = w.shape[2]
    gn_g, gn_be, Wu2, bu2, gu2, beu2 = tailp
    cout = Wu2.shape[1]
    cp = max(c, 128) + 128 + (max(cout, 128) if mxg is not None else 0)
    q = max(8, min(512, (4_000_000 // (H * cp * 4)) // 8 * 8))
    if c <= 64 and n > 2500:
        q = min(q, 80)
    if n <= q:
        q = -(-n // 8) * 8
    n_pad = -(-n // q) * q
    grid = n_pad // q
    npf = _pad3(npf, n_pad)
    nff = _pad3(nff, n_pad)
    qp16 = _pad_rows(qp16, n_pad)
    if mxg is not None:
        mxg = _pad3(mxg, n_pad)
    kpx = jnp.full((3, 16), 1e6, F32).at[:, :KS].set(kp.T)
    inv_sig = 1.0 / sigma
    small = c <= 64

    igm = _ig_mats(mid)
    igc = _ig_mats(cout)
    args = [npf, nff, qp16, kpx]
    specs = [
        pl.BlockSpec((q, H, 16), lambda i: (i, 0, 0)),
        pl.BlockSpec((q, H, c), lambda i: (i, 0, 0)),
        pl.BlockSpec((q, 16), lambda i: (i, 0)),
        pl.BlockSpec((3, 16), lambda i: (0, 0)),
    ]
    if small:
        kc = jnp.arange(KS * c)
        e1 = (kc[None, :] // c == jnp.arange(16)[:, None]).astype(F32)
        e2 = (kc[None, :] % c == jnp.arange(c)[:, None]).astype(F32)
        args += [w.reshape(KS * c, mid), e1, e2]
        specs += [pl.BlockSpec((KS * c, mid), lambda i: (0, 0)),
                  pl.BlockSpec((16, KS * c), lambda i: (0, 0)),
                  pl.BlockSpec((c, KS * c), lambda i: (0, 0))]
    else:
        args += [w]
        specs += [pl.BlockSpec((KS, c, mid), lambda i: (0, 0, 0))]

    def full2(a):
        a = jnp.asarray(a)
        specs.append(pl.BlockSpec(a.shape, lambda i: (0,) * a.ndim))
        args.append(a)

    if mxg is not None:
        specs.append(pl.BlockSpec((q, H, cout), lambda i: (i, 0, 0)))
        args.append(mxg)
    else:
        full2(scf)
    for a in (_row2(gn_g), _row2(gn_be), Wu2, _row2(bu2), _row2(gu2),
              _row2(beu2), igm[0], igm[1], igc[0], igc[1]):
        full2(a)
    outs = [jax.ShapeDtypeStruct((n, cout), F32)]
    out_specs = [pl.BlockSpec((n, cout), lambda i: (0, 0))]
    if u1p is not None:
        w1, b1, g1, be1 = u1p
        ig1 = _ig_mats(w1.shape[1])
        for a in (w1, _row2(b1), _row2(g1), _row2(be1), ig1[0], ig1[1]):
            full2(a)
        outs.append(jax.ShapeDtypeStruct((n, w1.shape[1]), F32))
        out_specs.append(pl.BlockSpec((n, w1.shape[1]), lambda i: (0, 0)))
    if scp is not None:
        w2, b2, g2, be2 = scp
        ig2 = _ig_mats(w2.shape[1])
        for a in (w2, _row2(b2), _row2(g2), _row2(be2), ig2[0], ig2[1]):
            full2(a)
        outs.append(jax.ShapeDtypeStruct((n, w2.shape[1]), F32))
        out_specs.append(pl.BlockSpec((n, w2.shape[1]), lambda i: (0, 0)))

    u1_ws = u1p[0].shape[1] if u1p is not None else None
    sc_ws = scp[0].shape[1] if scp is not None else None
    scratch = [pltpu.VMEM((n_pad, mid), F32),
               pltpu.VMEM((n_pad, cout), F32)]
    if mxg is not None:
        scratch.append(pltpu.VMEM((n_pad, cout), F32))
    if u1p is not None:
        scratch.append(pltpu.VMEM((n_pad, u1_ws), F32))
    if scp is not None:
        scratch.append(pltpu.VMEM((n_pad, sc_ws), F32))

    n_in = len(args)
    n_out = len(outs)

    def body(*refs):
        i = pl.program_id(0)
        pos = 4
        np_r, nf_r, qp_r, kpx_r = refs[:4]
        if small:
            wf_r, e1_r, e2_r = refs[pos:pos + 3]
            pos += 3
        else:
            wf_r = refs[pos]
            pos += 1
        if mxg is not None:
            mx_r = refs[pos]
        else:
            scf_r = refs[pos]
        pos += 1
        (gn_g_r, gn_be_r, wu2_r, bu2_r, gu2_r, beu2_r,
         igm0_r, igm1_r, igc0_r, igc1_r) = refs[pos:pos + 10]
        pos += 10
        if u1p is not None:
            u1_rs = refs[pos:pos + 6]
            pos += 6
        if scp is not None:
            sc_rs = refs[pos:pos + 6]
            pos += 6
        orefs = refs[n_in:n_in + n_out]
        spos = n_in + n_out
        acc_r = refs[spos]
        u2s_r = refs[spos + 1]
        spos += 2
        if mxg is not None:
            mxs_r = refs[spos]
            spos += 1
        if u1p is not None:
            t1s_r = refs[spos]
            spos += 1
        if scp is not None:
            t2s_r = refs[spos]

        qpb = qp_r[...]
        kx = kpx_r[0:1, :]
        ky = kpx_r[1:2, :]
        kz = kpx_r[2:3, :]

        def weights(h):
            rel = np_r[:, h, :] - qpb
            dx = rel[:, 0:1] - kx
            dy = rel[:, 1:2] - ky
            dz = rel[:, 2:3] - kz
            d2 = dx * dx + dy * dy + dz * dz
            return jnp.maximum(1.0 - jnp.sqrt(d2) * inv_sig, 0.0)

        cnt = None
        for h in range(H):
            ns = (jnp.sum(nf_r[:, h, :], axis=1, keepdims=True) > 0.0
                  ).astype(F32)
            cnt = ns if cnt is None else cnt + ns
        if small:
            A = None
            for h in range(H):
                ww = jnp.dot(weights(h), e1_r[...],
                             preferred_element_type=F32)
                nw = jnp.dot(nf_r[:, h, :], e2_r[...],
                             preferred_element_type=F32)
                u = ww * nw
                A = u if A is None else A + u
            acc = jnp.dot(A, wf_r[...], preferred_element_type=F32)
        else:
            wlist = [weights(h) for h in range(H)]
            acc = None
            for k in range(KS):
                ak = None
                for h in range(H):
                    u = wlist[h][:, k:k + 1] * nf_r[:, h, :]
                    ak = u if ak is None else ak + u
                pk = jnp.dot(ak, wf_r[k], preferred_element_type=F32)
                acc = pk if acc is None else acc + pk
        acc_r[pl.ds(i * q, q), :] = acc / jnp.maximum(cnt, 1.0)
        if mxg is not None:
            m = mx_r[:, 0, :]
            for h in range(1, H):
                m = jnp.maximum(m, mx_r[:, h, :])
            mxs_r[pl.ds(i * q, q), :] = m

        @pl.when(i == grid - 1)
        def _tail():
            tq = min(1024, max(8, (n // 8) * 8))
            full = n // tq
            rem = n - full * tq

            def gn_coefs(s1, s2, ig0_v, ig1_v, cnt_, g_v, be_v):
                m_ = jnp.dot(s1, ig0_v, preferred_element_type=F32) / cnt_
                v = (jnp.dot(s2, ig0_v, preferred_element_type=F32) / cnt_
                     - m_ * m_)
                inv = lax.rsqrt(v + 1e-5)
                mc = jnp.dot(m_, ig1_v, preferred_element_type=F32)
                invc = jnp.dot(inv, ig1_v, preferred_element_type=F32)
                mult = invc * g_v
                return mult, be_v - mc * mult

            def p1(j, cy):
                t = acc_r[pl.ds(j * tq, tq), :]
                return (cy[0] + jnp.sum(t, 0, keepdims=True),
                        cy[1] + jnp.sum(t * t, 0, keepdims=True))
            s1, s2 = lax.fori_loop(0, full, p1,
                                   (jnp.zeros((1, mid), F32),
                                    jnp.zeros((1, mid), F32)))
            if rem:
                t = acc_r[pl.ds(full * tq, rem), :]
                s1 += jnp.sum(t, 0, keepdims=True)
                s2 += jnp.sum(t * t, 0, keepdims=True)
            mult1, add1 = gn_coefs(s1, s2, igm0_r[...], igm1_r[...],
                                   n * (mid // G), gn_g_r[...], gn_be_r[...])

            def p2c(j, sz):
                t = _leaky(acc_r[pl.ds(j * tq, sz), :] * mult1 + add1)
                return jnp.dot(t, wu2_r[...],
                               preferred_element_type=F32) + bu2_r[...]

            def p2(j, cy):
                u = p2c(j, tq)
                u2s_r[pl.ds(j * tq, tq), :] = u
                return (cy[0] + jnp.sum(u, 0, keepdims=True),
                        cy[1] + jnp.sum(u * u, 0, keepdims=True))
            s1, s2 = lax.fori_loop(0, full, p2,
                                   (jnp.zeros((1, cout), F32),
                                    jnp.zeros((1, cout), F32)))
            if rem:
                u = p2c(full, rem)
                u2s_r[pl.ds(full * tq, rem), :] = u
                s1 += jnp.sum(u, 0, keepdims=True)
                s2 += jnp.sum(u * u, 0, keepdims=True)
            mult2, add2 = gn_coefs(s1, s2, igc0_r[...], igc1_r[...],
                                   n * (cout // G), gu2_r[...], beu2_r[...])

            nu1 = u1p is not None
            nsc = scp is not None
            c1 = u1_ws if nu1 else 8
            c2 = sc_ws if nsc else 8

            def p3body(j, sz, cy):
                u = u2s_r[pl.ds(j * tq, sz), :]
                if mxg is not None:
                    sc_c = mxs_r[pl.ds(j * tq, sz), :]
                else:
                    sc_c = scf_r[pl.ds(j * tq, sz), :]
                f = _leaky(u * mult2 + add2 + sc_c)
                orefs[0][pl.ds(j * tq, sz), :] = f
                a1, b1_, a2, b2_ = cy
                if nu1:
                    t1 = (jnp.dot(f, u1_rs[0][...],
                                  preferred_element_type=F32) + u1_rs[1][...])
                    t1s_r[pl.ds(j * tq, sz), :] = t1
                    a1 = a1 + jnp.sum(t1, 0, keepdims=True)
                    b1_ = b1_ + jnp.sum(t1 * t1, 0, keepdims=True)
                if nsc:
                    t2 = (jnp.dot(f, sc_rs[0][...],
                                  preferred_element_type=F32) + sc_rs[1][...])
                    t2s_r[pl.ds(j * tq, sz), :] = t2
                    a2 = a2 + jnp.sum(t2, 0, keepdims=True)
                    b2_ = b2_ + jnp.sum(t2 * t2, 0, keepdims=True)
                return (a1, b1_, a2, b2_)

            cy0 = (jnp.zeros((1, c1), F32), jnp.zeros((1, c1), F32),
                   jnp.zeros((1, c2), F32), jnp.zeros((1, c2), F32))
            cy = lax.fori_loop(0, full, lambda j, cy: p3body(j, tq, cy), cy0)
            if rem:
                cy = p3body(full, rem, cy)
            a1, b1_, a2, b2_ = cy

            oi = 1
            if nu1:
                mu, au = gn_coefs(a1, b1_, u1_rs[4][...], u1_rs[5][...],
                                  n * (c1 // G), u1_rs[2][...], u1_rs[3][...])
                o1 = orefs[1]

                def p4(j, z):
                    o1[pl.ds(j * tq, tq), :] = _leaky(
                        t1s_r[pl.ds(j * tq, tq), :] * mu + au)
                    return z
                lax.fori_loop(0, full, p4, 0)
                if rem:
                    o1[pl.ds(full * tq, rem), :] = _leaky(
                        t1s_r[pl.ds(full * tq, rem), :] * mu + au)
                oi = 2
            if nsc:
                ms, as_ = gn_coefs(a2, b2_, sc_rs[4][...], sc_rs[5][...],
                                   n * (c2 // G), sc_rs[2][...], sc_rs[3][...])
                o2 = orefs[oi]

                def p5(j, z):
                    o2[pl.ds(j * tq, tq), :] = (
                        t2s_r[pl.ds(j * tq, tq), :] * ms + as_)
                    return z
                lax.fori_loop(0, full, p5, 0)
                if rem:
                    o2[pl.ds(full * tq, rem), :] = (
                        t2s_r[pl.ds(full * tq, rem), :] * ms + as_)

    res = pl.pallas_call(
        body,
        grid=(grid,),
        in_specs=specs,
        out_specs=out_specs,
        out_shape=outs,
        scratch_shapes=scratch)(*args)
    return res


def _first_block(npf, qp16, kp, w, sigma, g0, be0, u1p, scp):
    n = qp16.shape[0]
    d = w.shape[2]
    q = max(8, min(512, (4_000_000 // (H * 256 * 4)) // 8 * 8))
    if n <= q:
        q = -(-n // 8) * 8
    n_pad = -(-n // q) * q
    grid = n_pad // q
    npf = _pad3(npf, n_pad)
    qp16 = _pad_rows(qp16, n_pad)
    kpx = jnp.full((3, 16), 1e6, F32).at[:, :KS].set(kp.T)
    w0 = jnp.zeros((16, d), F32).at[:KS, :].set(w[:, 0, :])
    inv_sig = 1.0 / sigma
    ig0 = _ig_mats(d)
    w1, b1, g1, be1 = u1p
    ig1 = _ig_mats(w1.shape[1])
    w2, b2, g2, be2 = scp
    ig2 = _ig_mats(w2.shape[1])
    c1 = w1.shape[1]
    c2 = w2.shape[1]

    def body(np_r, qp_r, kpx_r, w0_r, g0_r, be0_r, ig00_r, ig01_r,
             w1_r, b1_r, g1_r, be1_r, ig10_r, ig11_r,
             w2_r, b2_r, g2_r, be2_r, ig20_r, ig21_r,
             o1_r, o2_r, acc_r, t1s_r, t2s_r):
        i = pl.program_id(0)
        qpb = qp_r[...]
        kx = kpx_r[0:1, :]
        ky = kpx_r[1:2, :]
        kz = kpx_r[2:3, :]
        S = None
        for h in range(H):
            rel = np_r[:, h, :] - qpb
            dx = rel[:, 0:1] - kx
            dy = rel[:, 1:2] - ky
            dz = rel[:, 2:3] - kz
            d2 = dx * dx + dy * dy + dz * dz
            wts = jnp.maximum(1.0 - jnp.sqrt(d2) * inv_sig, 0.0)
            S = wts if S is None else S + wts
        acc_r[pl.ds(i * q, q), :] = (
            jnp.dot(S, w0_r[...], preferred_element_type=F32) * (1.0 / H))

        @pl.when(i == grid - 1)
        def _tail():
            tq = min(1024, max(8, (n // 8) * 8))
            full = n // tq
            rem = n - full * tq

            def gn_coefs(s1, s2, ig0_v, ig1_v, cnt_, g_v, be_v):
                m_ = jnp.dot(s1, ig0_v, preferred_element_type=F32) / cnt_
                v = (jnp.dot(s2, ig0_v, preferred_element_type=F32) / cnt_
                     - m_ * m_)
                inv = lax.rsqrt(v + 1e-5)
                mc = jnp.dot(m_, ig1_v, preferred_element_type=F32)
                invc = jnp.dot(inv, ig1_v, preferred_element_type=F32)
                mult = invc * g_v
                return mult, be_v - mc * mult

            def p1(j, cy):
                t = acc_r[pl.ds(j * tq, tq), :]
                return (cy[0] + jnp.sum(t, 0, keepdims=True),
                        cy[1] + jnp.sum(t * t, 0, keepdims=True))
            s1, s2 = lax.fori_loop(0, full, p1,
                                   (jnp.zeros((1, d), F32),
                                    jnp.zeros((1, d), F32)))
            if rem:
                t = acc_r[pl.ds(full * tq, rem), :]
                s1 += jnp.sum(t, 0, keepdims=True)
                s2 += jnp.sum(t * t, 0, keepdims=True)
            mult0, add0 = gn_coefs(s1, s2, ig00_r[...], ig01_r[...],
                                   n * (d // G), g0_r[...], be0_r[...])

            def p2body(j, sz, cy):
                f = _leaky(acc_r[pl.ds(j * tq, sz), :] * mult0 + add0)
                t1 = jnp.dot(f, w1_r[...],
                             preferred_element_type=F32) + b1_r[...]
                t2 = jnp.dot(f, w2_r[...],
                             preferred_element_type=F32) + b2_r[...]
                t1s_r[pl.ds(j * tq, sz), :] = t1
                t2s_r[pl.ds(j * tq, sz), :] = t2
                a1, b1_, a2, b2_ = cy
                return (a1 + jnp.sum(t1, 0, keepdims=True),
                        b1_ + jnp.sum(t1 * t1, 0, keepdims=True),
                        a2 + jnp.sum(t2, 0, keepdims=True),
                        b2_ + jnp.sum(t2 * t2, 0, keepdims=True))
            cy0 = (jnp.zeros((1, c1), F32), jnp.zeros((1, c1), F32),
                   jnp.zeros((1, c2), F32), jnp.zeros((1, c2), F32))
            cy = lax.fori_loop(0, full, lambda j, cy: p2body(j, tq, cy), cy0)
            if rem:
                cy = p2body(full, rem, cy)
            a1, b1_, a2, b2_ = cy

            mu, au = gn_coefs(a1, b1_, ig10_r[...], ig11_r[...],
                              n * (c1 // G), g1_r[...], be1_r[...])

            def p3(j, z):
                o1_r[pl.ds(j * tq, tq), :] = _leaky(
                    t1s_r[pl.ds(j * tq, tq), :] * mu + au)
                return z
            lax.fori_loop(0, full, p3, 0)
            if rem:
                o1_r[pl.ds(full * tq, rem), :] = _leaky(
                    t1s_r[pl.ds(full * tq, rem), :] * mu + au)

            ms, as_ = gn_coefs(a2, b2_, ig20_r[...], ig21_r[...],
                               n * (c2 // G), g2_r[...], be2_r[...])

            def p4(j, z):
                o2_r[pl.ds(j * tq, tq), :] = (
                    t2s_r[pl.ds(j * tq, tq), :] * ms + as_)
                return z
            lax.fori_loop(0, full, p4, 0)
            if rem:
                o2_r[pl.ds(full * tq, rem), :] = (
                    t2s_r[pl.ds(full * tq, rem), :] * ms + as_)

    full_spec = lambda a: pl.BlockSpec(a.shape, lambda i: (0,) * a.ndim)
    args = [npf, qp16, kpx, w0, _row2(g0), _row2(be0), ig0[0], ig0[1],
            w1, _row2(b1), _row2(g1), _row2(be1), ig1[0], ig1[1],
            w2, _row2(b2), _row2(g2), _row2(be2), ig2[0], ig2[1]]
    specs = ([pl.BlockSpec((q, H, 16), lambda i: (i, 0, 0)),
              pl.BlockSpec((q, 16), lambda i: (i, 0))]
             + [full_spec(a) for a in args[2:]])
    return pl.pallas_call(
        body,
        grid=(grid,),
        in_specs=specs,
        out_specs=[pl.BlockSpec((n, c1), lambda i: (0, 0)),
                   pl.BlockSpec((n, c2), lambda i: (0, 0))],
        out_shape=[jax.ShapeDtypeStruct((n, c1), F32),
                   jax.ShapeDtypeStruct((n, c2), F32)],
        scratch_shapes=[pltpu.VMEM((n_pad, d), F32),
                        pltpu.VMEM((n_pad, c1), F32),
                        pltpu.VMEM((n_pad, c2), F32)])(*args)


def kernel(feats, points_0, points_1, points_2, points_3, neighbors_0,
           neighbors_1, neighbors_2, neighbors_3, subsampling_0,
           subsampling_1, subsampling_2, upsampling_0, upsampling_1,
           upsampling_2, params):
    p = params

    def up(name):
        return (p[name + '_W'], p[name + '_b'], p[name + '_g'],
                p[name + '_be'])

    def tp(name):
        return (p[name + '_n_g'], p[name + '_n_be'], p[name + '_u2_W'],
                p[name + '_u2_b'], p[name + '_u2_g'], p[name + '_u2_be'])

    n1, n2, n3, n4 = (points_0.shape[0], points_1.shape[0],
                      points_2.shape[0], points_3.shape[0])
    pt0, pt1, pt2, pt3 = (_pad16(points_0), _pad16(points_1),
                          _pad16(points_2), _pad16(points_3))

    nb0 = neighbors_0.reshape(-1)
    nb1 = neighbors_1.reshape(-1)
    nb2 = neighbors_2.reshape(-1)
    nb3 = neighbors_3.reshape(-1)
    sb0 = subsampling_0.reshape(-1)
    sb1 = subsampling_1.reshape(-1)
    sb2 = subsampling_2.reshape(-1)

    pts_all = jnp.concatenate([pt0, pt1, pt2, pt3], axis=0)
    o1, o2, o3 = n1, n1 + n2, n1 + n2 + n3
    idx_all = jnp.concatenate([
        nb0, sb0, nb1 + o1, sb1 + o1, nb2 + o2, sb2 + o2, nb3 + o3])
    gp = _gather_rows(pts_all, idx_all)
    s = 0
    segs = []
    for cnt, nn in ((n1 * H, n1), (n2 * H, n2), (n2 * H, n2), (n3 * H, n3),
                    (n3 * H, n3), (n4 * H, n4), (n4 * H, n4)):
        segs.append(gp[s:s + cnt].reshape(nn, H, 16))
        s += cnt
    np_n0, np_s0, np_n1, np_s1, np_n2, np_s2, np_n3 = segs

    h12, sc12 = _first_block(np_n0, pt0, p['e11_kp'], p['e11_w'], S0,
                             p['e11_g'], p['e11_be'],
                             up('e12_u1'), up('e12_sc'))
    f1, h21 = _block(np_n0, _gather_rows(h12, nb0).reshape(n1, H, 32),
                     pt0, p['e12_kp'], p['e12_w'], S0, tp('e12'),
                     scf=sc12, u1p=up('e21_u1'))

    mxg = _gather_rows(f1, sb0).reshape(n2, H, 128)
    f2a, h22, sc22 = _block(np_s0, _gather_rows(h21, sb0).reshape(n2, H, 32),
                            pt1, p['e21_kp'], p['e21_w'], S0, tp('e21'),
                            mxg=mxg, u1p=up('e22_u1'), scp=up('e22_sc'))
    f2b, h23 = _block(np_n1, _gather_rows(h22, nb1).reshape(n2, H, 64),
                      pt1, p['e22_kp'], p['e22_w'], 2 * S0, tp('e22'),
                      scf=sc22, u1p=up('e23_u1'))
    f2, h31 = _block(np_n1, _gather_rows(h23, nb1).reshape(n2, H, 64),
                     pt1, p['e23_kp'], p['e23_w'], 2 * S0, tp('e23'),
                     scf=f2b, u1p=up('e31_u1'))

    mxg = _gather_rows(f2, sb1).reshape(n3, H, 256)
    f3a, h32, sc32 = _block(np_s1, _gather_rows(h31, sb1).reshape(n3, H, 64),
                            pt2, p['e31_kp'], p['e31_w'], 2 * S0, tp('e31'),
                            mxg=mxg, u1p=up('e32_u1'), scp=up('e32_sc'))
    f3b, h33 = _block(np_n2, _gather_rows(h32, nb2).reshape(n3, H, 128),
                      pt2, p['e32_kp'], p['e32_w'], 4 * S0, tp('e32'),
                      scf=sc32, u1p=up('e33_u1'))
    f3, h41 = _block(np_n2, _gather_rows(h33, nb2).reshape(n3, H, 128),
                     pt2, p['e33_kp'], p['e33_w'], 4 * S0, tp('e33'),
                     scf=f3b, u1p=up('e41_u1'))

    mxg = _gather_rows(f3, sb2).reshape(n4, H, 512)
    f4a, h42, sc42 = _block(np_s2, _gather_rows(h41, sb2).reshape(n4, H, 128),
                            pt3, p['e41_kp'], p['e41_w'], 4 * S0, tp('e41'),
                            mxg=mxg, u1p=up('e42_u1'), scp=up('e42_sc'))
    f4b, h43 = _block(np_n3, _gather_rows(h42, nb3).reshape(n4, H, 256),
                      pt3, p['e42_kp'], p['e42_w'], 8 * S0, tp('e42'),
                      scf=sc42, u1p=up('e43_u1'))
    (f4,) = _block(np_n3, _gather_rows(h43, nb3).reshape(n4, H, 256),
                   pt3, p['e43_kp'], p['e43_w'], 8 * S0, tp('e43'),
                   scf=f4b)

    up3 = _gather_rows(f4, upsampling_2[:, 0])
    l3 = _unary2(up3, f3, p['d3_W'][:1024], p['d3_W'][1024:], p['d3_b'],
                 p['d3_g'], p['d3_be'])
    up2 = _gather_rows(l3, upsampling_1[:, 0])
    l2 = _unary2(up2, f2, p['d2_W'][:512], p['d2_W'][512:], p['d2_b'],
                 relu=False)

    return ([l2, l3, f4], [f1, f2, f3])


# speedup vs baseline: 1.0258x; 1.0173x over previous
import functools

import jax
import jax.numpy as jnp
from jax import lax
from jax.experimental import pallas as pl
from jax.experimental.pallas import tpu as pltpu
from jax.experimental.pallas import tpu_sc as plsc

F32 = jnp.float32
H = 32
G = 32
KS = 15
S0 = 2.0


def _gather_rows(table, idx):
    V, D = table.shape
    B = idx.shape[0]
    info = plsc.get_sparse_core_info()
    nw = info.num_cores * info.num_subcores
    ch_max = min(128, (65536 // D) // 8 * 8)
    r = -(-B // nw)
    if r <= ch_max:
        ch = max(8, -(-r // 8) * 8)
        r = ch
    else:
        ch = ch_max
        r = -(-r // ch) * ch
    bp = r * nw
    nch = r // ch
    idx = jnp.pad(idx.astype(jnp.int32).reshape(-1), (0, bp - B))

    mesh = plsc.VectorSubcoreMesh(core_axis_name="c", subcore_axis_name="s")

    @functools.partial(
        pl.kernel, mesh=mesh,
        compiler_params=pltpu.CompilerParams(use_tc_tiling_on_sc=False),
        out_type=jax.ShapeDtypeStruct((bp, D), F32),
        scratch_types=[
            pltpu.VMEM((ch,), jnp.int32),
            pltpu.VMEM((ch, D), F32),
            pltpu.SemaphoreType.DMA,
        ],
    )
    def gk(table_hbm, idx_hbm, out_hbm, idx_v, rows_v, sem):
        wid = lax.axis_index("s") * info.num_cores + lax.axis_index("c")
        base = wid * r

        def one(start):
            pltpu.sync_copy(idx_hbm.at[pl.ds(start, ch)], idx_v)
            pltpu.async_copy(table_hbm.at[idx_v], rows_v, sem).wait()
            pltpu.sync_copy(rows_v, out_hbm.at[pl.ds(start, ch)])

        if nch <= 8:
            for j in range(nch):
                one(pl.multiple_of(base + j * ch, 8))
        else:
            def body(j, carry):
                one(pl.multiple_of(base + j * ch, 8))
                return carry
            lax.fori_loop(0, nch, body, 0)

    return gk(table, idx)[:B]


def _leaky(x):
    return jnp.maximum(x, 0.1 * x)


def _gn_vals(t, g, be, ig, igt, n):
    c = t.shape[1]
    cnt = n * (c // G)
    m = (jnp.sum(t, axis=0, keepdims=True) @ ig) * (1.0 / cnt)
    mc = m @ igt
    tc = t - mc
    v = (jnp.sum(tc * tc, axis=0, keepdims=True) @ ig) * (1.0 / cnt)
    inv = lax.rsqrt(v + 1e-5) @ igt
    return tc * inv * g + be


def _ig_mats(c):
    a = (jnp.arange(c)[:, None] // (c // G) == jnp.arange(G)[None, :])
    ig = a.astype(F32)
    return ig, ig.T


def _row2(x):
    return x.reshape(1, -1)


def _pad_rows(x, n_pad):
    n = x.shape[0]
    if n_pad == n:
        return x
    return jnp.pad(x, ((0, n_pad - n), (0, 0)))


def _pad3(x, n_pad):
    n = x.shape[0]
    if n_pad == n:
        return x
    return jnp.pad(x, ((0, n_pad - n), (0, 0), (0, 0)))


def _pad16(pts):
    n = pts.shape[0]
    return jnp.concatenate([pts, jnp.zeros((n, 13), F32)], axis=1)


def _unary2(x1, x2, W1, W2, b, g=None, be=None, relu=True):
    n = x1.shape[0]
    cout = W1.shape[1]
    gn = g is not None
    if gn:
        ig, igt = _ig_mats(cout)

    def body(*refs):
        if gn:
            x1_r, x2_r, w1_r, w2_r, b_r, g_r, be_r, ig_r, igt_r, o_r = refs
        else:
            x1_r, x2_r, w1_r, w2_r, b_r, o_r = refs
        t = (jnp.dot(x1_r[...], w1_r[...], preferred_element_type=F32)
             + jnp.dot(x2_r[...], w2_r[...], preferred_element_type=F32)
             + b_r[...])
        if gn:
            t = _gn_vals(t, g_r[...], be_r[...], ig_r[...], igt_r[...], n)
        if relu:
            t = _leaky(t)
        o_r[...] = t

    args = [x1, x2, W1, W2, _row2(b)]
    if gn:
        args += [_row2(g), _row2(be), ig, igt]
    return pl.pallas_call(
        body, out_shape=jax.ShapeDtypeStruct((n, cout), F32))(*args)


def _block(npf, nff, qp16, kp, w, sigma, tailp, scf=None, mxg=None,
           u1p=None, scp=None):
    n = qp16.shape[0]
    mid = w.shape[2]
    gn_g, gn_be, Wu2, bu2, gu2, beu2 = tailp
    cout = Wu2.shape[1]
    c = nff.shape[2] - (cout if mxg is not None else 0)
    coff = cout if mxg is not None else 0
    cp = max(c + coff, 128) + 128
    q = max(8, min(512, (4_000_000 // (H * cp * 4)) // 8 * 8))
    if c <= 64 and n > 2500:
        q = min(q, 80)
    if n <= q:
        q = -(-n // 8) * 8
    n_pad = -(-n // q) * q
    grid = n_pad // q
    npf = _pad3(npf, n_pad)
    nff = _pad3(nff, n_pad)
    qp16 = _pad_rows(qp16, n_pad)
    kpx = jnp.full((3, 16), 1e6, F32).at[:, :KS].set(kp.T)
    inv_sig = 1.0 / sigma
    small = c <= 64

    igm = _ig_mats(mid)
    igc = _ig_mats(cout)
    args = [npf, nff, qp16, kpx]
    specs = [
        pl.BlockSpec((q, H, 16), lambda i: (i, 0, 0)),
        pl.BlockSpec((q, H, c + coff), lambda i: (i, 0, 0)),
        pl.BlockSpec((q, 16), lambda i: (i, 0)),
        pl.BlockSpec((3, 16), lambda i: (0, 0)),
    ]
    if small:
        kc = jnp.arange(KS * c)
        e1 = (kc[None, :] // c == jnp.arange(16)[:, None]).astype(F32)
        e2 = (kc[None, :] % c == jnp.arange(c)[:, None]).astype(F32)
        args += [w.reshape(KS * c, mid), e1, e2]
        specs += [pl.BlockSpec((KS * c, mid), lambda i: (0, 0)),
                  pl.BlockSpec((16, KS * c), lambda i: (0, 0)),
                  pl.BlockSpec((c, KS * c), lambda i: (0, 0))]
    else:
        args += [w]
        specs += [pl.BlockSpec((KS, c, mid), lambda i: (0, 0, 0))]

    def full2(a):
        a = jnp.asarray(a)
        specs.append(pl.BlockSpec(a.shape, lambda i: (0,) * a.ndim))
        args.append(a)

    if mxg is None:
        full2(scf)
    for a in (_row2(gn_g), _row2(gn_be), Wu2, _row2(bu2), _row2(gu2),
              _row2(beu2), igm[0], igm[1], igc[0], igc[1]):
        full2(a)
    outs = [jax.ShapeDtypeStruct((n, cout), F32)]
    out_specs = [pl.BlockSpec((n, cout), lambda i: (0, 0))]
    if u1p is not None:
        w1, b1, g1, be1 = u1p
        ig1 = _ig_mats(w1.shape[1])
        for a in (w1, _row2(b1), _row2(g1), _row2(be1), ig1[0], ig1[1]):
            full2(a)
        outs.append(jax.ShapeDtypeStruct((n, w1.shape[1]), F32))
        out_specs.append(pl.BlockSpec((n, w1.shape[1]), lambda i: (0, 0)))
    if scp is not None:
        w2, b2, g2, be2 = scp
        ig2 = _ig_mats(w2.shape[1])
        for a in (w2, _row2(b2), _row2(g2), _row2(be2), ig2[0], ig2[1]):
            full2(a)
        outs.append(jax.ShapeDtypeStruct((n, w2.shape[1]), F32))
        out_specs.append(pl.BlockSpec((n, w2.shape[1]), lambda i: (0, 0)))

    u1_ws = u1p[0].shape[1] if u1p is not None else None
    sc_ws = scp[0].shape[1] if scp is not None else None
    scratch = [pltpu.VMEM((n_pad, mid), F32),
               pltpu.VMEM((n_pad, cout), F32)]
    if mxg is not None:
        scratch.append(pltpu.VMEM((n_pad, cout), F32))
    if u1p is not None:
        scratch.append(pltpu.VMEM((n_pad, u1_ws), F32))
    if scp is not None:
        scratch.append(pltpu.VMEM((n_pad, sc_ws), F32))

    n_in = len(args)
    n_out = len(outs)

    def body(*refs):
        i = pl.program_id(0)
        pos = 4
        np_r, nf_r, qp_r, kpx_r = refs[:4]
        if small:
            wf_r, e1_r, e2_r = refs[pos:pos + 3]
            pos += 3
        else:
            wf_r = refs[pos]
            pos += 1
        if mxg is None:
            scf_r = refs[pos]
            pos += 1
        (gn_g_r, gn_be_r, wu2_r, bu2_r, gu2_r, beu2_r,
         igm0_r, igm1_r, igc0_r, igc1_r) = refs[pos:pos + 10]
        pos += 10
        if u1p is not None:
            u1_rs = refs[pos:pos + 6]
            pos += 6
        if scp is not None:
            sc_rs = refs[pos:pos + 6]
            pos += 6
        orefs = refs[n_in:n_in + n_out]
        spos = n_in + n_out
        acc_r = refs[spos]
        u2s_r = refs[spos + 1]
        spos += 2
        if mxg is not None:
            mxs_r = refs[spos]
            spos += 1
        if u1p is not None:
            t1s_r = refs[spos]
            spos += 1
        if scp is not None:
            t2s_r = refs[spos]

        qpb = qp_r[...]
        kx = kpx_r[0:1, :]
        ky = kpx_r[1:2, :]
        kz = kpx_r[2:3, :]

        def weights(h):
            rel = np_r[:, h, :] - qpb
            dx = rel[:, 0:1] - kx
            dy = rel[:, 1:2] - ky
            dz = rel[:, 2:3] - kz
            d2 = dx * dx + dy * dy + dz * dz
            return jnp.maximum(1.0 - jnp.sqrt(d2) * inv_sig, 0.0)

        cnt = None
        for h in range(H):
            ns = (jnp.sum(nf_r[:, h, coff:coff + c], axis=1, keepdims=True) > 0.0
                  ).astype(F32)
            cnt = ns if cnt is None else cnt + ns
        if small:
            A = None
            for h in range(H):
                ww = jnp.dot(weights(h), e1_r[...],
                             preferred_element_type=F32)
                nw = jnp.dot(nf_r[:, h, coff:coff + c], e2_r[...],
                             preferred_element_type=F32)
                u = ww * nw
                A = u if A is None else A + u
            acc = jnp.dot(A, wf_r[...], preferred_element_type=F32)
        else:
            wlist = [weights(h) for h in range(H)]
            acc = None
            for k in range(KS):
                ak = None
                for h in range(H):
                    u = wlist[h][:, k:k + 1] * nf_r[:, h, coff:coff + c]
                    ak = u if ak is None else ak + u
                pk = jnp.dot(ak, wf_r[k], preferred_element_type=F32)
                acc = pk if acc is None else acc + pk
        acc_r[pl.ds(i * q, q), :] = acc / jnp.maximum(cnt, 1.0)
        if mxg is not None:
            m = nf_r[:, 0, 0:cout]
            for h in range(1, H):
                m = jnp.maximum(m, nf_r[:, h, 0:cout])
            mxs_r[pl.ds(i * q, q), :] = m

        @pl.when(i == grid - 1)
        def _tail():
            tq = min(1024, max(8, (n // 8) * 8))
            full = n // tq
            rem = n - full * tq

            def gn_coefs(s1, s2, ig0_v, ig1_v, cnt_, g_v, be_v):
                m_ = jnp.dot(s1, ig0_v, preferred_element_type=F32) / cnt_
                v = (jnp.dot(s2, ig0_v, preferred_element_type=F32) / cnt_
                     - m_ * m_)
                inv = lax.rsqrt(v + 1e-5)
                mc = jnp.dot(m_, ig1_v, preferred_element_type=F32)
                invc = jnp.dot(inv, ig1_v, preferred_element_type=F32)
                mult = invc * g_v
                return mult, be_v - mc * mult

            def p1(j, cy):
                t = acc_r[pl.ds(j * tq, tq), :]
                return (cy[0] + jnp.sum(t, 0, keepdims=True),
                        cy[1] + jnp.sum(t * t, 0, keepdims=True))
            s1, s2 = lax.fori_loop(0, full, p1,
                                   (jnp.zeros((1, mid), F32),
                                    jnp.zeros((1, mid), F32)))
            if rem:
                t = acc_r[pl.ds(full * tq, rem), :]
                s1 += jnp.sum(t, 0, keepdims=True)
                s2 += jnp.sum(t * t, 0, keepdims=True)
            mult1, add1 = gn_coefs(s1, s2, igm0_r[...], igm1_r[...],
                                   n * (mid // G), gn_g_r[...], gn_be_r[...])

            def p2c(j, sz):
                t = _leaky(acc_r[pl.ds(j * tq, sz), :] * mult1 + add1)
                return jnp.dot(t, wu2_r[...],
                               preferred_element_type=F32) + bu2_r[...]

            def p2(j, cy):
                u = p2c(j, tq)
                u2s_r[pl.ds(j * tq, tq), :] = u
                return (cy[0] + jnp.sum(u, 0, keepdims=True),
                        cy[1] + jnp.sum(u * u, 0, keepdims=True))
            s1, s2 = lax.fori_loop(0, full, p2,
                                   (jnp.zeros((1, cout), F32),
                                    jnp.zeros((1, cout), F32)))
            if rem:
                u = p2c(full, rem)
                u2s_r[pl.ds(full * tq, rem), :] = u
                s1 += jnp.sum(u, 0, keepdims=True)
                s2 += jnp.sum(u * u, 0, keepdims=True)
            mult2, add2 = gn_coefs(s1, s2, igc0_r[...], igc1_r[...],
                                   n * (cout // G), gu2_r[...], beu2_r[...])

            nu1 = u1p is not None
            nsc = scp is not None
            c1 = u1_ws if nu1 else 8
            c2 = sc_ws if nsc else 8

            def p3body(j, sz, cy):
                u = u2s_r[pl.ds(j * tq, sz), :]
                if mxg is not None:
                    sc_c = mxs_r[pl.ds(j * tq, sz), :]
                else:
                    sc_c = scf_r[pl.ds(j * tq, sz), :]
                f = _leaky(u * mult2 + add2 + sc_c)
                orefs[0][pl.ds(j * tq, sz), :] = f
                a1, b1_, a2, b2_ = cy
                if nu1:
                    t1 = (jnp.dot(f, u1_rs[0][...],
                                  preferred_element_type=F32) + u1_rs[1][...])
                    t1s_r[pl.ds(j * tq, sz), :] = t1
                    a1 = a1 + jnp.sum(t1, 0, keepdims=True)
                    b1_ = b1_ + jnp.sum(t1 * t1, 0, keepdims=True)
                if nsc:
                    t2 = (jnp.dot(f, sc_rs[0][...],
                                  preferred_element_type=F32) + sc_rs[1][...])
                    t2s_r[pl.ds(j * tq, sz), :] = t2
                    a2 = a2 + jnp.sum(t2, 0, keepdims=True)
                    b2_ = b2_ + jnp.sum(t2 * t2, 0, keepdims=True)
                return (a1, b1_, a2, b2_)

            cy0 = (jnp.zeros((1, c1), F32), jnp.zeros((1, c1), F32),
                   jnp.zeros((1, c2), F32), jnp.zeros((1, c2), F32))
            cy = lax.fori_loop(0, full, lambda j, cy: p3body(j, tq, cy), cy0)
            if rem:
                cy = p3body(full, rem, cy)
            a1, b1_, a2, b2_ = cy

            oi = 1
            if nu1:
                mu, au = gn_coefs(a1, b1_, u1_rs[4][...], u1_rs[5][...],
                                  n * (c1 // G), u1_rs[2][...], u1_rs[3][...])
                o1 = orefs[1]

                def p4(j, z):
                    o1[pl.ds(j * tq, tq), :] = _leaky(
                        t1s_r[pl.ds(j * tq, tq), :] * mu + au)
                    return z
                lax.fori_loop(0, full, p4, 0)
                if rem:
                    o1[pl.ds(full * tq, rem), :] = _leaky(
                        t1s_r[pl.ds(full * tq, rem), :] * mu + au)
                oi = 2
            if nsc:
                ms, as_ = gn_coefs(a2, b2_, sc_rs[4][...], sc_rs[5][...],
                                   n * (c2 // G), sc_rs[2][...], sc_rs[3][...])
                o2 = orefs[oi]

                def p5(j, z):
                    o2[pl.ds(j * tq, tq), :] = (
                        t2s_r[pl.ds(j * tq, tq), :] * ms + as_)
                    return z
                lax.fori_loop(0, full, p5, 0)
                if rem:
                    o2[pl.ds(full * tq, rem), :] = (
                        t2s_r[pl.ds(full * tq, rem), :] * ms + as_)

    res = pl.pallas_call(
        body,
        grid=(grid,),
        in_specs=specs,
        out_specs=out_specs,
        out_shape=outs,
        scratch_shapes=scratch)(*args)
    return res


def _first_block(npf, qp16, kp, w, sigma, g0, be0, u1p, scp):
    n = qp16.shape[0]
    d = w.shape[2]
    q = max(8, min(512, (4_000_000 // (H * 256 * 4)) // 8 * 8))
    if n <= q:
        q = -(-n // 8) * 8
    n_pad = -(-n // q) * q
    grid = n_pad // q
    npf = _pad3(npf, n_pad)
    qp16 = _pad_rows(qp16, n_pad)
    kpx = jnp.full((3, 16), 1e6, F32).at[:, :KS].set(kp.T)
    w0 = jnp.zeros((16, d), F32).at[:KS, :].set(w[:, 0, :])
    inv_sig = 1.0 / sigma
    ig0 = _ig_mats(d)
    w1, b1, g1, be1 = u1p
    ig1 = _ig_mats(w1.shape[1])
    w2, b2, g2, be2 = scp
    ig2 = _ig_mats(w2.shape[1])
    c1 = w1.shape[1]
    c2 = w2.shape[1]

    def body(np_r, qp_r, kpx_r, w0_r, g0_r, be0_r, ig00_r, ig01_r,
             w1_r, b1_r, g1_r, be1_r, ig10_r, ig11_r,
             w2_r, b2_r, g2_r, be2_r, ig20_r, ig21_r,
             o1_r, o2_r, acc_r, t1s_r, t2s_r):
        i = pl.program_id(0)
        qpb = qp_r[...]
        kx = kpx_r[0:1, :]
        ky = kpx_r[1:2, :]
        kz = kpx_r[2:3, :]
        S = None
        for h in range(H):
            rel = np_r[:, h, :] - qpb
            dx = rel[:, 0:1] - kx
            dy = rel[:, 1:2] - ky
            dz = rel[:, 2:3] - kz
            d2 = dx * dx + dy * dy + dz * dz
            wts = jnp.maximum(1.0 - jnp.sqrt(d2) * inv_sig, 0.0)
            S = wts if S is None else S + wts
        acc_r[pl.ds(i * q, q), :] = (
            jnp.dot(S, w0_r[...], preferred_element_type=F32) * (1.0 / H))

        @pl.when(i == grid - 1)
        def _tail():
            tq = min(1024, max(8, (n // 8) * 8))
            full = n // tq
            rem = n - full * tq

            def gn_coefs(s1, s2, ig0_v, ig1_v, cnt_, g_v, be_v):
                m_ = jnp.dot(s1, ig0_v, preferred_element_type=F32) / cnt_
                v = (jnp.dot(s2, ig0_v, preferred_element_type=F32) / cnt_
                     - m_ * m_)
                inv = lax.rsqrt(v + 1e-5)
                mc = jnp.dot(m_, ig1_v, preferred_element_type=F32)
                invc = jnp.dot(inv, ig1_v, preferred_element_type=F32)
                mult = invc * g_v
                return mult, be_v - mc * mult

            def p1(j, cy):
                t = acc_r[pl.ds(j * tq, tq), :]
                return (cy[0] + jnp.sum(t, 0, keepdims=True),
                        cy[1] + jnp.sum(t * t, 0, keepdims=True))
            s1, s2 = lax.fori_loop(0, full, p1,
                                   (jnp.zeros((1, d), F32),
                                    jnp.zeros((1, d), F32)))
            if rem:
                t = acc_r[pl.ds(full * tq, rem), :]
                s1 += jnp.sum(t, 0, keepdims=True)
                s2 += jnp.sum(t * t, 0, keepdims=True)
            mult0, add0 = gn_coefs(s1, s2, ig00_r[...], ig01_r[...],
                                   n * (d // G), g0_r[...], be0_r[...])

            def p2body(j, sz, cy):
                f = _leaky(acc_r[pl.ds(j * tq, sz), :] * mult0 + add0)
                t1 = jnp.dot(f, w1_r[...],
                             preferred_element_type=F32) + b1_r[...]
                t2 = jnp.dot(f, w2_r[...],
                             preferred_element_type=F32) + b2_r[...]
                t1s_r[pl.ds(j * tq, sz), :] = t1
                t2s_r[pl.ds(j * tq, sz), :] = t2
                a1, b1_, a2, b2_ = cy
                return (a1 + jnp.sum(t1, 0, keepdims=True),
                        b1_ + jnp.sum(t1 * t1, 0, keepdims=True),
                        a2 + jnp.sum(t2, 0, keepdims=True),
                        b2_ + jnp.sum(t2 * t2, 0, keepdims=True))
            cy0 = (jnp.zeros((1, c1), F32), jnp.zeros((1, c1), F32),
                   jnp.zeros((1, c2), F32), jnp.zeros((1, c2), F32))
            cy = lax.fori_loop(0, full, lambda j, cy: p2body(j, tq, cy), cy0)
            if rem:
                cy = p2body(full, rem, cy)
            a1, b1_, a2, b2_ = cy

            mu, au = gn_coefs(a1, b1_, ig10_r[...], ig11_r[...],
                              n * (c1 // G), g1_r[...], be1_r[...])

            def p3(j, z):
                o1_r[pl.ds(j * tq, tq), :] = _leaky(
                    t1s_r[pl.ds(j * tq, tq), :] * mu + au)
                return z
            lax.fori_loop(0, full, p3, 0)
            if rem:
                o1_r[pl.ds(full * tq, rem), :] = _leaky(
                    t1s_r[pl.ds(full * tq, rem), :] * mu + au)

            ms, as_ = gn_coefs(a2, b2_, ig20_r[...], ig21_r[...],
                               n * (c2 // G), g2_r[...], be2_r[...])

            def p4(j, z):
                o2_r[pl.ds(j * tq, tq), :] = (
                    t2s_r[pl.ds(j * tq, tq), :] * ms + as_)
                return z
            lax.fori_loop(0, full, p4, 0)
            if rem:
                o2_r[pl.ds(full * tq, rem), :] = (
                    t2s_r[pl.ds(full * tq, rem), :] * ms + as_)

    full_spec = lambda a: pl.BlockSpec(a.shape, lambda i: (0,) * a.ndim)
    args = [npf, qp16, kpx, w0, _row2(g0), _row2(be0), ig0[0], ig0[1],
            w1, _row2(b1), _row2(g1), _row2(be1), ig1[0], ig1[1],
            w2, _row2(b2), _row2(g2), _row2(be2), ig2[0], ig2[1]]
    specs = ([pl.BlockSpec((q, H, 16), lambda i: (i, 0, 0)),
              pl.BlockSpec((q, 16), lambda i: (i, 0))]
             + [full_spec(a) for a in args[2:]])
    return pl.pallas_call(
        body,
        grid=(grid,),
        in_specs=specs,
        out_specs=[pl.BlockSpec((n, c1), lambda i: (0, 0)),
                   pl.BlockSpec((n, c2), lambda i: (0, 0))],
        out_shape=[jax.ShapeDtypeStruct((n, c1), F32),
                   jax.ShapeDtypeStruct((n, c2), F32)],
        scratch_shapes=[pltpu.VMEM((n_pad, d), F32),
                        pltpu.VMEM((n_pad, c1), F32),
                        pltpu.VMEM((n_pad, c2), F32)])(*args)


def kernel(feats, points_0, points_1, points_2, points_3, neighbors_0,
           neighbors_1, neighbors_2, neighbors_3, subsampling_0,
           subsampling_1, subsampling_2, upsampling_0, upsampling_1,
           upsampling_2, params):
    p = params

    def up(name):
        return (p[name + '_W'], p[name + '_b'], p[name + '_g'],
                p[name + '_be'])

    def tp(name):
        return (p[name + '_n_g'], p[name + '_n_be'], p[name + '_u2_W'],
                p[name + '_u2_b'], p[name + '_u2_g'], p[name + '_u2_be'])

    n1, n2, n3, n4 = (points_0.shape[0], points_1.shape[0],
                      points_2.shape[0], points_3.shape[0])
    pt0, pt1, pt2, pt3 = (_pad16(points_0), _pad16(points_1),
                          _pad16(points_2), _pad16(points_3))

    nb0 = neighbors_0.reshape(-1)
    nb1 = neighbors_1.reshape(-1)
    nb2 = neighbors_2.reshape(-1)
    nb3 = neighbors_3.reshape(-1)
    sb0 = subsampling_0.reshape(-1)
    sb1 = subsampling_1.reshape(-1)
    sb2 = subsampling_2.reshape(-1)

    pts_all = jnp.concatenate([pt0, pt1, pt2, pt3], axis=0)
    o1, o2, o3 = n1, n1 + n2, n1 + n2 + n3
    idx_all = jnp.concatenate([
        nb0, sb0, nb1 + o1, sb1 + o1, nb2 + o2, sb2 + o2, nb3 + o3])
    gp = _gather_rows(pts_all, idx_all)
    s = 0
    segs = []
    for cnt, nn in ((n1 * H, n1), (n2 * H, n2), (n2 * H, n2), (n3 * H, n3),
                    (n3 * H, n3), (n4 * H, n4), (n4 * H, n4)):
        segs.append(gp[s:s + cnt].reshape(nn, H, 16))
        s += cnt
    np_n0, np_s0, np_n1, np_s1, np_n2, np_s2, np_n3 = segs

    h12, sc12 = _first_block(np_n0, pt0, p['e11_kp'], p['e11_w'], S0,
                             p['e11_g'], p['e11_be'],
                             up('e12_u1'), up('e12_sc'))
    f1, h21 = _block(np_n0, _gather_rows(h12, nb0).reshape(n1, H, 32),
                     pt0, p['e12_kp'], p['e12_w'], S0, tp('e12'),
                     scf=sc12, u1p=up('e21_u1'))

    g21 = _gather_rows(jnp.concatenate([f1, h21], axis=1),
                       sb0).reshape(n2, H, 160)
    f2a, h22, sc22 = _block(np_s0, g21,
                            pt1, p['e21_kp'], p['e21_w'], S0, tp('e21'),
                            mxg=True, u1p=up('e22_u1'), scp=up('e22_sc'))
    f2b, h23 = _block(np_n1, _gather_rows(h22, nb1).reshape(n2, H, 64),
                      pt1, p['e22_kp'], p['e22_w'], 2 * S0, tp('e22'),
                      scf=sc22, u1p=up('e23_u1'))
    f2, h31 = _block(np_n1, _gather_rows(h23, nb1).reshape(n2, H, 64),
                     pt1, p['e23_kp'], p['e23_w'], 2 * S0, tp('e23'),
                     scf=f2b, u1p=up('e31_u1'))

    g31 = _gather_rows(jnp.concatenate([f2, h31], axis=1),
                       sb1).reshape(n3, H, 320)
    f3a, h32, sc32 = _block(np_s1, g31,
                            pt2, p['e31_kp'], p['e31_w'], 2 * S0, tp('e31'),
                            mxg=True, u1p=up('e32_u1'), scp=up('e32_sc'))
    f3b, h33 = _block(np_n2, _gather_rows(h32, nb2).reshape(n3, H, 128),
                      pt2, p['e32_kp'], p['e32_w'], 4 * S0, tp('e32'),
                      scf=sc32, u1p=up('e33_u1'))
    f3, h41 = _block(np_n2, _gather_rows(h33, nb2).reshape(n3, H, 128),
                     pt2, p['e33_kp'], p['e33_w'], 4 * S0, tp('e33'),
                     scf=f3b, u1p=up('e41_u1'))

    g41 = _gather_rows(jnp.concatenate([f3, h41], axis=1),
                       sb2).reshape(n4, H, 640)
    f4a, h42, sc42 = _block(np_s2, g41,
                            pt3, p['e41_kp'], p['e41_w'], 4 * S0, tp('e41'),
                            mxg=True, u1p=up('e42_u1'), scp=up('e42_sc'))
    f4b, h43 = _block(np_n3, _gather_rows(h42, nb3).reshape(n4, H, 256),
                      pt3, p['e42_kp'], p['e42_w'], 8 * S0, tp('e42'),
                      scf=sc42, u1p=up('e43_u1'))
    (f4,) = _block(np_n3, _gather_rows(h43, nb3).reshape(n4, H, 256),
                   pt3, p['e43_kp'], p['e43_w'], 8 * S0, tp('e43'),
                   scf=f4b)

    up3 = _gather_rows(f4, upsampling_2[:, 0])
    l3 = _unary2(up3, f3, p['d3_W'][:1024], p['d3_W'][1024:], p['d3_b'],
                 p['d3_g'], p['d3_be'])
    up2 = _gather_rows(l3, upsampling_1[:, 0])
    l2 = _unary2(up2, f2, p['d2_W'][:512], p['d2_W'][512:], p['d2_b'],
                 relu=False)

    return ([l2, l3, f4], [f1, f2, f3])


# speedup vs baseline: 1.0356x; 1.0095x over previous
import functools

import jax
import jax.numpy as jnp
from jax import lax
from jax.experimental import pallas as pl
from jax.experimental.pallas import tpu as pltpu
from jax.experimental.pallas import tpu_sc as plsc

F32 = jnp.float32
H = 32
G = 32
KS = 15
S0 = 2.0


def _gather_rows(table, idx):
    V, D = table.shape
    B = idx.shape[0]
    info = plsc.get_sparse_core_info()
    nw = info.num_cores * info.num_subcores
    ch_max = min(128, (65536 // D) // 8 * 8)
    r = -(-B // nw)
    if r <= ch_max:
        ch = max(8, -(-r // 8) * 8)
        r = ch
    else:
        ch = ch_max
        r = -(-r // ch) * ch
    bp = r * nw
    nch = r // ch
    idx = jnp.pad(idx.astype(jnp.int32).reshape(-1), (0, bp - B))

    mesh = plsc.VectorSubcoreMesh(core_axis_name="c", subcore_axis_name="s")

    @functools.partial(
        pl.kernel, mesh=mesh,
        compiler_params=pltpu.CompilerParams(use_tc_tiling_on_sc=False),
        out_type=jax.ShapeDtypeStruct((bp, D), F32),
        scratch_types=[
            pltpu.VMEM((r,), jnp.int32),
            pltpu.VMEM((ch, D), F32),
            pltpu.SemaphoreType.DMA,
        ],
    )
    def gk(table_hbm, idx_hbm, out_hbm, idx_v, rows_v, sem):
        wid = lax.axis_index("s") * info.num_cores + lax.axis_index("c")
        base = wid * r
        pltpu.sync_copy(idx_hbm.at[pl.ds(pl.multiple_of(base, 8), r)], idx_v)

        def one(j, start):
            pltpu.async_copy(table_hbm.at[idx_v.at[pl.ds(j * ch, ch)]],
                             rows_v, sem).wait()
            pltpu.sync_copy(rows_v, out_hbm.at[pl.ds(start, ch)])

        if nch <= 8:
            for j in range(nch):
                one(j, pl.multiple_of(base + j * ch, 8))
        else:
            def body(j, carry):
                one(j, pl.multiple_of(base + j * ch, 8))
                return carry
            lax.fori_loop(0, nch, body, 0)

    return gk(table, idx)[:B]


def _leaky(x):
    return jnp.maximum(x, 0.1 * x)


def _gn_vals(t, g, be, ig, igt, n):
    c = t.shape[1]
    cnt = n * (c // G)
    m = (jnp.sum(t, axis=0, keepdims=True) @ ig) * (1.0 / cnt)
    mc = m @ igt
    tc = t - mc
    v = (jnp.sum(tc * tc, axis=0, keepdims=True) @ ig) * (1.0 / cnt)
    inv = lax.rsqrt(v + 1e-5) @ igt
    return tc * inv * g + be


def _ig_mats(c):
    a = (jnp.arange(c)[:, None] // (c // G) == jnp.arange(G)[None, :])
    ig = a.astype(F32)
    return ig, ig.T


def _row2(x):
    return x.reshape(1, -1)


def _pad_rows(x, n_pad):
    n = x.shape[0]
    if n_pad == n:
        return x
    return jnp.pad(x, ((0, n_pad - n), (0, 0)))


def _pad3(x, n_pad):
    n = x.shape[0]
    if n_pad == n:
        return x
    return jnp.pad(x, ((0, n_pad - n), (0, 0), (0, 0)))


def _pad16(pts):
    n = pts.shape[0]
    return jnp.concatenate([pts, jnp.zeros((n, 13), F32)], axis=1)


def _unary2(x1, x2, W1, W2, b, g=None, be=None, relu=True):
    n = x1.shape[0]
    cout = W1.shape[1]
    gn = g is not None
    if gn:
        ig, igt = _ig_mats(cout)

    def body(*refs):
        if gn:
            x1_r, x2_r, w1_r, w2_r, b_r, g_r, be_r, ig_r, igt_r, o_r = refs
        else:
            x1_r, x2_r, w1_r, w2_r, b_r, o_r = refs
        t = (jnp.dot(x1_r[...], w1_r[...], preferred_element_type=F32)
             + jnp.dot(x2_r[...], w2_r[...], preferred_element_type=F32)
             + b_r[...])
        if gn:
            t = _gn_vals(t, g_r[...], be_r[...], ig_r[...], igt_r[...], n)
        if relu:
            t = _leaky(t)
        o_r[...] = t

    args = [x1, x2, W1, W2, _row2(b)]
    if gn:
        args += [_row2(g), _row2(be), ig, igt]
    return pl.pallas_call(
        body, out_shape=jax.ShapeDtypeStruct((n, cout), F32))(*args)


def _block(npf, nff, qp16, kp, w, sigma, tailp, scf=None, mxg=None,
           u1p=None, scp=None):
    n = qp16.shape[0]
    mid = w.shape[2]
    gn_g, gn_be, Wu2, bu2, gu2, beu2 = tailp
    cout = Wu2.shape[1]
    c = nff.shape[2] - (cout if mxg is not None else 0)
    coff = cout if mxg is not None else 0
    cp = max(c + coff, 128) + 128
    q = max(8, min(512, (4_000_000 // (H * cp * 4)) // 8 * 8))
    if c <= 64 and n > 2500:
        q = min(q, 80)
    if n <= q:
        q = -(-n // 8) * 8
    n_pad = -(-n // q) * q
    grid = n_pad // q
    npf = _pad3(npf, n_pad)
    nff = _pad3(nff, n_pad)
    qp16 = _pad_rows(qp16, n_pad)
    kpx = jnp.full((3, 16), 1e6, F32).at[:, :KS].set(kp.T)
    inv_sig = 1.0 / sigma
    small = c <= 64

    igm = _ig_mats(mid)
    igc = _ig_mats(cout)
    args = [npf, nff, qp16, kpx]
    specs = [
        pl.BlockSpec((q, H, 16), lambda i: (i, 0, 0)),
        pl.BlockSpec((q, H, c + coff), lambda i: (i, 0, 0)),
        pl.BlockSpec((q, 16), lambda i: (i, 0)),
        pl.BlockSpec((3, 16), lambda i: (0, 0)),
    ]
    if small:
        kc = jnp.arange(KS * c)
        e1 = (kc[None, :] // c == jnp.arange(16)[:, None]).astype(F32)
        e2 = (kc[None, :] % c == jnp.arange(c)[:, None]).astype(F32)
        args += [w.reshape(KS * c, mid), e1, e2]
        specs += [pl.BlockSpec((KS * c, mid), lambda i: (0, 0)),
                  pl.BlockSpec((16, KS * c), lambda i: (0, 0)),
                  pl.BlockSpec((c, KS * c), lambda i: (0, 0))]
    else:
        args += [w]
        specs += [pl.BlockSpec((KS, c, mid), lambda i: (0, 0, 0))]

    def full2(a):
        a = jnp.asarray(a)
        specs.append(pl.BlockSpec(a.shape, lambda i: (0,) * a.ndim))
        args.append(a)

    if mxg is None:
        full2(scf)
    for a in (_row2(gn_g), _row2(gn_be), Wu2, _row2(bu2), _row2(gu2),
              _row2(beu2), igm[0], igm[1], igc[0], igc[1]):
        full2(a)
    outs = [jax.ShapeDtypeStruct((n, cout), F32)]
    out_specs = [pl.BlockSpec((n, cout), lambda i: (0, 0))]
    if u1p is not None:
        w1, b1, g1, be1 = u1p
        ig1 = _ig_mats(w1.shape[1])
        for a in (w1, _row2(b1), _row2(g1), _row2(be1), ig1[0], ig1[1]):
            full2(a)
        outs.append(jax.ShapeDtypeStruct((n, w1.shape[1]), F32))
        out_specs.append(pl.BlockSpec((n, w1.shape[1]), lambda i: (0, 0)))
    if scp is not None:
        w2, b2, g2, be2 = scp
        ig2 = _ig_mats(w2.shape[1])
        for a in (w2, _row2(b2), _row2(g2), _row2(be2), ig2[0], ig2[1]):
            full2(a)
        outs.append(jax.ShapeDtypeStruct((n, w2.shape[1]), F32))
        out_specs.append(pl.BlockSpec((n, w2.shape[1]), lambda i: (0, 0)))

    u1_ws = u1p[0].shape[1] if u1p is not None else None
    sc_ws = scp[0].shape[1] if scp is not None else None
    scratch = [pltpu.VMEM((n_pad, mid), F32),
               pltpu.VMEM((n_pad, cout), F32)]
    if mxg is not None:
        scratch.append(pltpu.VMEM((n_pad, cout), F32))
    if u1p is not None:
        scratch.append(pltpu.VMEM((n_pad, u1_ws), F32))
    if scp is not None:
        scratch.append(pltpu.VMEM((n_pad, sc_ws), F32))

    n_in = len(args)
    n_out = len(outs)

    def body(*refs):
        i = pl.program_id(0)
        pos = 4
        np_r, nf_r, qp_r, kpx_r = refs[:4]
        if small:
            wf_r, e1_r, e2_r = refs[pos:pos + 3]
            pos += 3
        else:
            wf_r = refs[pos]
            pos += 1
        if mxg is None:
            scf_r = refs[pos]
            pos += 1
        (gn_g_r, gn_be_r, wu2_r, bu2_r, gu2_r, beu2_r,
         igm0_r, igm1_r, igc0_r, igc1_r) = refs[pos:pos + 10]
        pos += 10
        if u1p is not None:
            u1_rs = refs[pos:pos + 6]
            pos += 6
        if scp is not None:
            sc_rs = refs[pos:pos + 6]
            pos += 6
        orefs = refs[n_in:n_in + n_out]
        spos = n_in + n_out
        acc_r = refs[spos]
        u2s_r = refs[spos + 1]
        spos += 2
        if mxg is not None:
            mxs_r = refs[spos]
            spos += 1
        if u1p is not None:
            t1s_r = refs[spos]
            spos += 1
        if scp is not None:
            t2s_r = refs[spos]

        qpb = qp_r[...]
        kx = kpx_r[0:1, :]
        ky = kpx_r[1:2, :]
        kz = kpx_r[2:3, :]

        def weights(h):
            rel = np_r[:, h, :] - qpb
            dx = rel[:, 0:1] - kx
            dy = rel[:, 1:2] - ky
            dz = rel[:, 2:3] - kz
            d2 = dx * dx + dy * dy + dz * dz
            return jnp.maximum(1.0 - jnp.sqrt(d2) * inv_sig, 0.0)

        cnt = None
        for h in range(H):
            ns = (jnp.sum(nf_r[:, h, coff:coff + c], axis=1, keepdims=True) > 0.0
                  ).astype(F32)
            cnt = ns if cnt is None else cnt + ns
        if small:
            A = None
            for h in range(H):
                ww = jnp.dot(weights(h), e1_r[...],
                             preferred_element_type=F32)
                nw = jnp.dot(nf_r[:, h, coff:coff + c], e2_r[...],
                             preferred_element_type=F32)
                u = ww * nw
                A = u if A is None else A + u
            acc = jnp.dot(A, wf_r[...], preferred_element_type=F32)
        else:
            wlist = [weights(h) for h in range(H)]
            acc = None
            for k in range(KS):
                ak = None
                for h in range(H):
                    u = wlist[h][:, k:k + 1] * nf_r[:, h, coff:coff + c]
                    ak = u if ak is None else ak + u
                pk = jnp.dot(ak, wf_r[k], preferred_element_type=F32)
                acc = pk if acc is None else acc + pk
        acc_r[pl.ds(i * q, q), :] = acc / jnp.maximum(cnt, 1.0)
        if mxg is not None:
            m = nf_r[:, 0, 0:cout]
            for h in range(1, H):
                m = jnp.maximum(m, nf_r[:, h, 0:cout])
            mxs_r[pl.ds(i * q, q), :] = m

        @pl.when(i == grid - 1)
        def _tail():
            tq = min(1024, max(8, (n // 8) * 8))
            full = n // tq
            rem = n - full * tq

            def gn_coefs(s1, s2, ig0_v, ig1_v, cnt_, g_v, be_v):
                m_ = jnp.dot(s1, ig0_v, preferred_element_type=F32) / cnt_
                v = (jnp.dot(s2, ig0_v, preferred_element_type=F32) / cnt_
                     - m_ * m_)
                inv = lax.rsqrt(v + 1e-5)
                mc = jnp.dot(m_, ig1_v, preferred_element_type=F32)
                invc = jnp.dot(inv, ig1_v, preferred_element_type=F32)
                mult = invc * g_v
                return mult, be_v - mc * mult

            def p1(j, cy):
                t = acc_r[pl.ds(j * tq, tq), :]
                return (cy[0] + jnp.sum(t, 0, keepdims=True),
                        cy[1] + jnp.sum(t * t, 0, keepdims=True))
            s1, s2 = lax.fori_loop(0, full, p1,
                                   (jnp.zeros((1, mid), F32),
                                    jnp.zeros((1, mid), F32)))
            if rem:
                t = acc_r[pl.ds(full * tq, rem), :]
                s1 += jnp.sum(t, 0, keepdims=True)
                s2 += jnp.sum(t * t, 0, keepdims=True)
            mult1, add1 = gn_coefs(s1, s2, igm0_r[...], igm1_r[...],
                                   n * (mid // G), gn_g_r[...], gn_be_r[...])

            def p2c(j, sz):
                t = _leaky(acc_r[pl.ds(j * tq, sz), :] * mult1 + add1)
                return jnp.dot(t, wu2_r[...],
                               preferred_element_type=F32) + bu2_r[...]

            def p2(j, cy):
                u = p2c(j, tq)
                u2s_r[pl.ds(j * tq, tq), :] = u
                return (cy[0] + jnp.sum(u, 0, keepdims=True),
                        cy[1] + jnp.sum(u * u, 0, keepdims=True))
            s1, s2 = lax.fori_loop(0, full, p2,
                                   (jnp.zeros((1, cout), F32),
                                    jnp.zeros((1, cout), F32)))
            if rem:
                u = p2c(full, rem)
                u2s_r[pl.ds(full * tq, rem), :] = u
                s1 += jnp.sum(u, 0, keepdims=True)
                s2 += jnp.sum(u * u, 0, keepdims=True)
            mult2, add2 = gn_coefs(s1, s2, igc0_r[...], igc1_r[...],
                                   n * (cout // G), gu2_r[...], beu2_r[...])

            nu1 = u1p is not None
            nsc = scp is not None
            c1 = u1_ws if nu1 else 8
            c2 = sc_ws if nsc else 8

            def p3body(j, sz, cy):
                u = u2s_r[pl.ds(j * tq, sz), :]
                if mxg is not None:
                    sc_c = mxs_r[pl.ds(j * tq, sz), :]
                else:
                    sc_c = scf_r[pl.ds(j * tq, sz), :]
                f = _leaky(u * mult2 + add2 + sc_c)
                orefs[0][pl.ds(j * tq, sz), :] = f
                a1, b1_, a2, b2_ = cy
                if nu1:
                    t1 = (jnp.dot(f, u1_rs[0][...],
                                  preferred_element_type=F32) + u1_rs[1][...])
                    t1s_r[pl.ds(j * tq, sz), :] = t1
                    a1 = a1 + jnp.sum(t1, 0, keepdims=True)
                    b1_ = b1_ + jnp.sum(t1 * t1, 0, keepdims=True)
                if nsc:
                    t2 = (jnp.dot(f, sc_rs[0][...],
                                  preferred_element_type=F32) + sc_rs[1][...])
                    t2s_r[pl.ds(j * tq, sz), :] = t2
                    a2 = a2 + jnp.sum(t2, 0, keepdims=True)
                    b2_ = b2_ + jnp.sum(t2 * t2, 0, keepdims=True)
                return (a1, b1_, a2, b2_)

            cy0 = (jnp.zeros((1, c1), F32), jnp.zeros((1, c1), F32),
                   jnp.zeros((1, c2), F32), jnp.zeros((1, c2), F32))
            cy = lax.fori_loop(0, full, lambda j, cy: p3body(j, tq, cy), cy0)
            if rem:
                cy = p3body(full, rem, cy)
            a1, b1_, a2, b2_ = cy

            oi = 1
            if nu1:
                mu, au = gn_coefs(a1, b1_, u1_rs[4][...], u1_rs[5][...],
                                  n * (c1 // G), u1_rs[2][...], u1_rs[3][...])
                o1 = orefs[1]

                def p4(j, z):
                    o1[pl.ds(j * tq, tq), :] = _leaky(
                        t1s_r[pl.ds(j * tq, tq), :] * mu + au)
                    return z
                lax.fori_loop(0, full, p4, 0)
                if rem:
                    o1[pl.ds(full * tq, rem), :] = _leaky(
                        t1s_r[pl.ds(full * tq, rem), :] * mu + au)
                oi = 2
            if nsc:
                ms, as_ = gn_coefs(a2, b2_, sc_rs[4][...], sc_rs[5][...],
                                   n * (c2 // G), sc_rs[2][...], sc_rs[3][...])
                o2 = orefs[oi]

                def p5(j, z):
                    o2[pl.ds(j * tq, tq), :] = (
                        t2s_r[pl.ds(j * tq, tq), :] * ms + as_)
                    return z
                lax.fori_loop(0, full, p5, 0)
                if rem:
                    o2[pl.ds(full * tq, rem), :] = (
                        t2s_r[pl.ds(full * tq, rem), :] * ms + as_)

    res = pl.pallas_call(
        body,
        grid=(grid,),
        in_specs=specs,
        out_specs=out_specs,
        out_shape=outs,
        scratch_shapes=scratch)(*args)
    return res


def _first_block(npf, qp16, kp, w, sigma, g0, be0, u1p, scp):
    n = qp16.shape[0]
    d = w.shape[2]
    q = max(8, min(512, (4_000_000 // (H * 256 * 4)) // 8 * 8))
    if n <= q:
        q = -(-n // 8) * 8
    n_pad = -(-n // q) * q
    grid = n_pad // q
    npf = _pad3(npf, n_pad)
    qp16 = _pad_rows(qp16, n_pad)
    kpx = jnp.full((3, 16), 1e6, F32).at[:, :KS].set(kp.T)
    w0 = jnp.zeros((16, d), F32).at[:KS, :].set(w[:, 0, :])
    inv_sig = 1.0 / sigma
    ig0 = _ig_mats(d)
    w1, b1, g1, be1 = u1p
    ig1 = _ig_mats(w1.shape[1])
    w2, b2, g2, be2 = scp
    ig2 = _ig_mats(w2.shape[1])
    c1 = w1.shape[1]
    c2 = w2.shape[1]

    def body(np_r, qp_r, kpx_r, w0_r, g0_r, be0_r, ig00_r, ig01_r,
             w1_r, b1_r, g1_r, be1_r, ig10_r, ig11_r,
             w2_r, b2_r, g2_r, be2_r, ig20_r, ig21_r,
             o1_r, o2_r, acc_r, t1s_r, t2s_r):
        i = pl.program_id(0)
        qpb = qp_r[...]
        kx = kpx_r[0:1, :]
        ky = kpx_r[1:2, :]
        kz = kpx_r[2:3, :]
        S = None
        for h in range(H):
            rel = np_r[:, h, :] - qpb
            dx = rel[:, 0:1] - kx
            dy = rel[:, 1:2] - ky
            dz = rel[:, 2:3] - kz
            d2 = dx * dx + dy * dy + dz * dz
            wts = jnp.maximum(1.0 - jnp.sqrt(d2) * inv_sig, 0.0)
            S = wts if S is None else S + wts
        acc_r[pl.ds(i * q, q), :] = (
            jnp.dot(S, w0_r[...], preferred_element_type=F32) * (1.0 / H))

        @pl.when(i == grid - 1)
        def _tail():
            tq = min(1024, max(8, (n // 8) * 8))
            full = n // tq
            rem = n - full * tq

            def gn_coefs(s1, s2, ig0_v, ig1_v, cnt_, g_v, be_v):
                m_ = jnp.dot(s1, ig0_v, preferred_element_type=F32) / cnt_
                v = (jnp.dot(s2, ig0_v, preferred_element_type=F32) / cnt_
                     - m_ * m_)
                inv = lax.rsqrt(v + 1e-5)
                mc = jnp.dot(m_, ig1_v, preferred_element_type=F32)
                invc = jnp.dot(inv, ig1_v, preferred_element_type=F32)
                mult = invc * g_v
                return mult, be_v - mc * mult

            def p1(j, cy):
                t = acc_r[pl.ds(j * tq, tq), :]
                return (cy[0] + jnp.sum(t, 0, keepdims=True),
                        cy[1] + jnp.sum(t * t, 0, keepdims=True))
            s1, s2 = lax.fori_loop(0, full, p1,
                                   (jnp.zeros((1, d), F32),
                                    jnp.zeros((1, d), F32)))
            if rem:
                t = acc_r[pl.ds(full * tq, rem), :]
                s1 += jnp.sum(t, 0, keepdims=True)
                s2 += jnp.sum(t * t, 0, keepdims=True)
            mult0, add0 = gn_coefs(s1, s2, ig00_r[...], ig01_r[...],
                                   n * (d // G), g0_r[...], be0_r[...])

            def p2body(j, sz, cy):
                f = _leaky(acc_r[pl.ds(j * tq, sz), :] * mult0 + add0)
                t1 = jnp.dot(f, w1_r[...],
                             preferred_element_type=F32) + b1_r[...]
                t2 = jnp.dot(f, w2_r[...],
                             preferred_element_type=F32) + b2_r[...]
                t1s_r[pl.ds(j * tq, sz), :] = t1
                t2s_r[pl.ds(j * tq, sz), :] = t2
                a1, b1_, a2, b2_ = cy
                return (a1 + jnp.sum(t1, 0, keepdims=True),
                        b1_ + jnp.sum(t1 * t1, 0, keepdims=True),
                        a2 + jnp.sum(t2, 0, keepdims=True),
                        b2_ + jnp.sum(t2 * t2, 0, keepdims=True))
            cy0 = (jnp.zeros((1, c1), F32), jnp.zeros((1, c1), F32),
                   jnp.zeros((1, c2), F32), jnp.zeros((1, c2), F32))
            cy = lax.fori_loop(0, full, lambda j, cy: p2body(j, tq, cy), cy0)
            if rem:
                cy = p2body(full, rem, cy)
            a1, b1_, a2, b2_ = cy

            mu, au = gn_coefs(a1, b1_, ig10_r[...], ig11_r[...],
                              n * (c1 // G), g1_r[...], be1_r[...])

            def p3(j, z):
                o1_r[pl.ds(j * tq, tq), :] = _leaky(
                    t1s_r[pl.ds(j * tq, tq), :] * mu + au)
                return z
            lax.fori_loop(0, full, p3, 0)
            if rem:
                o1_r[pl.ds(full * tq, rem), :] = _leaky(
                    t1s_r[pl.ds(full * tq, rem), :] * mu + au)

            ms, as_ = gn_coefs(a2, b2_, ig20_r[...], ig21_r[...],
                               n * (c2 // G), g2_r[...], be2_r[...])

            def p4(j, z):
                o2_r[pl.ds(j * tq, tq), :] = (
                    t2s_r[pl.ds(j * tq, tq), :] * ms + as_)
                return z
            lax.fori_loop(0, full, p4, 0)
            if rem:
                o2_r[pl.ds(full * tq, rem), :] = (
                    t2s_r[pl.ds(full * tq, rem), :] * ms + as_)

    full_spec = lambda a: pl.BlockSpec(a.shape, lambda i: (0,) * a.ndim)
    args = [npf, qp16, kpx, w0, _row2(g0), _row2(be0), ig0[0], ig0[1],
            w1, _row2(b1), _row2(g1), _row2(be1), ig1[0], ig1[1],
            w2, _row2(b2), _row2(g2), _row2(be2), ig2[0], ig2[1]]
    specs = ([pl.BlockSpec((q, H, 16), lambda i: (i, 0, 0)),
              pl.BlockSpec((q, 16), lambda i: (i, 0))]
             + [full_spec(a) for a in args[2:]])
    return pl.pallas_call(
        body,
        grid=(grid,),
        in_specs=specs,
        out_specs=[pl.BlockSpec((n, c1), lambda i: (0, 0)),
                   pl.BlockSpec((n, c2), lambda i: (0, 0))],
        out_shape=[jax.ShapeDtypeStruct((n, c1), F32),
                   jax.ShapeDtypeStruct((n, c2), F32)],
        scratch_shapes=[pltpu.VMEM((n_pad, d), F32),
                        pltpu.VMEM((n_pad, c1), F32),
                        pltpu.VMEM((n_pad, c2), F32)])(*args)


def kernel(feats, points_0, points_1, points_2, points_3, neighbors_0,
           neighbors_1, neighbors_2, neighbors_3, subsampling_0,
           subsampling_1, subsampling_2, upsampling_0, upsampling_1,
           upsampling_2, params):
    p = params

    def up(name):
        return (p[name + '_W'], p[name + '_b'], p[name + '_g'],
                p[name + '_be'])

    def tp(name):
        return (p[name + '_n_g'], p[name + '_n_be'], p[name + '_u2_W'],
                p[name + '_u2_b'], p[name + '_u2_g'], p[name + '_u2_be'])

    n1, n2, n3, n4 = (points_0.shape[0], points_1.shape[0],
                      points_2.shape[0], points_3.shape[0])
    pt0, pt1, pt2, pt3 = (_pad16(points_0), _pad16(points_1),
                          _pad16(points_2), _pad16(points_3))

    nb0 = neighbors_0.reshape(-1)
    nb1 = neighbors_1.reshape(-1)
    nb2 = neighbors_2.reshape(-1)
    nb3 = neighbors_3.reshape(-1)
    sb0 = subsampling_0.reshape(-1)
    sb1 = subsampling_1.reshape(-1)
    sb2 = subsampling_2.reshape(-1)

    pts_all = jnp.concatenate([pt0, pt1, pt2, pt3], axis=0)
    o1, o2, o3 = n1, n1 + n2, n1 + n2 + n3
    idx_all = jnp.concatenate([
        nb0, sb0, nb1 + o1, sb1 + o1, nb2 + o2, sb2 + o2, nb3 + o3])
    gp = _gather_rows(pts_all, idx_all)
    s = 0
    segs = []
    for cnt, nn in ((n1 * H, n1), (n2 * H, n2), (n2 * H, n2), (n3 * H, n3),
                    (n3 * H, n3), (n4 * H, n4), (n4 * H, n4)):
        segs.append(gp[s:s + cnt].reshape(nn, H, 16))
        s += cnt
    np_n0, np_s0, np_n1, np_s1, np_n2, np_s2, np_n3 = segs

    h12, sc12 = _first_block(np_n0, pt0, p['e11_kp'], p['e11_w'], S0,
                             p['e11_g'], p['e11_be'],
                             up('e12_u1'), up('e12_sc'))
    f1, h21 = _block(np_n0, _gather_rows(h12, nb0).reshape(n1, H, 32),
                     pt0, p['e12_kp'], p['e12_w'], S0, tp('e12'),
                     scf=sc12, u1p=up('e21_u1'))

    g21 = _gather_rows(jnp.concatenate([f1, h21], axis=1),
                       sb0).reshape(n2, H, 160)
    f2a, h22, sc22 = _block(np_s0, g21,
                            pt1, p['e21_kp'], p['e21_w'], S0, tp('e21'),
                            mxg=True, u1p=up('e22_u1'), scp=up('e22_sc'))
    f2b, h23 = _block(np_n1, _gather_rows(h22, nb1).reshape(n2, H, 64),
                      pt1, p['e22_kp'], p['e22_w'], 2 * S0, tp('e22'),
                      scf=sc22, u1p=up('e23_u1'))
    f2, h31 = _block(np_n1, _gather_rows(h23, nb1).reshape(n2, H, 64),
                     pt1, p['e23_kp'], p['e23_w'], 2 * S0, tp('e23'),
                     scf=f2b, u1p=up('e31_u1'))

    g31 = _gather_rows(jnp.concatenate([f2, h31], axis=1),
                       sb1).reshape(n3, H, 320)
    f3a, h32, sc32 = _block(np_s1, g31,
                            pt2, p['e31_kp'], p['e31_w'], 2 * S0, tp('e31'),
                            mxg=True, u1p=up('e32_u1'), scp=up('e32_sc'))
    f3b, h33 = _block(np_n2, _gather_rows(h32, nb2).reshape(n3, H, 128),
                      pt2, p['e32_kp'], p['e32_w'], 4 * S0, tp('e32'),
                      scf=sc32, u1p=up('e33_u1'))
    f3, h41 = _block(np_n2, _gather_rows(h33, nb2).reshape(n3, H, 128),
                     pt2, p['e33_kp'], p['e33_w'], 4 * S0, tp('e33'),
                     scf=f3b, u1p=up('e41_u1'))

    g41 = _gather_rows(jnp.concatenate([f3, h41], axis=1),
                       sb2).reshape(n4, H, 640)
    f4a, h42, sc42 = _block(np_s2, g41,
                            pt3, p['e41_kp'], p['e41_w'], 4 * S0, tp('e41'),
                            mxg=True, u1p=up('e42_u1'), scp=up('e42_sc'))
    f4b, h43 = _block(np_n3, _gather_rows(h42, nb3).reshape(n4, H, 256),
                      pt3, p['e42_kp'], p['e42_w'], 8 * S0, tp('e42'),
                      scf=sc42, u1p=up('e43_u1'))
    (f4,) = _block(np_n3, _gather_rows(h43, nb3).reshape(n4, H, 256),
                   pt3, p['e43_kp'], p['e43_w'], 8 * S0, tp('e43'),
                   scf=f4b)

    up3 = _gather_rows(f4, upsampling_2[:, 0])
    l3 = _unary2(up3, f3, p['d3_W'][:1024], p['d3_W'][1024:], p['d3_b'],
                 p['d3_g'], p['d3_be'])
    up2 = _gather_rows(l3, upsampling_1[:, 0])
    l2 = _unary2(up2, f2, p['d2_W'][:512], p['d2_W'][512:], p['d2_b'],
                 relu=False)

    return ([l2, l3, f4], [f1, f2, f3])
